# baseline (speedup 1.0000x reference)
import functools

import jax
import jax.numpy as jnp
from jax import lax
from jax.experimental import pallas as pl
from jax.experimental.pallas import tpu as pltpu
from jax.experimental.pallas import tpu_sc as plsc

_RB = 512
_PW = 80


def _gru_body(x_ref, wt0_ref, wb0_ref, b0_ref,
              wt1_ref, wb1_ref, b1_ref, out_ref, h1_ref, h2_ref):
    t = pl.program_id(0)
    nt = pl.num_programs(0)

    @pl.when(t == 0)
    def _():
        h1_ref[...] = jnp.zeros_like(h1_ref)
        h2_ref[...] = jnp.zeros_like(h2_ref)

    xt = x_ref[0]
    h1 = h1_ref[...]
    h2 = h2_ref[...]
    hh = h1.shape[1]

    def cell(xpart, h, wt, wb, b):
        g = jnp.dot(xpart, wt) + jnp.dot(h, wb) + b
        rz = jax.nn.sigmoid(g[:, :2 * hh])
        r = rz[:, :hh]
        z = rz[:, hh:]
        n = jnp.tanh(g[:, 2 * hh:3 * hh] + r * g[:, 3 * hh:])
        return z * (h - n) + n

    h1n = cell(xt, h1, wt0_ref[...], wb0_ref[...], b0_ref[...])
    h2n = cell(h1n, h2, wt1_ref[...], wb1_ref[...], b1_ref[...])
    h1_ref[...] = h1n
    h2_ref[...] = h2n

    @pl.when(t == nt - 1)
    def _():
        out_ref[...] = h2n


def _gate_weights(w_ih, w_hh, b_ih, b_hh, dpad):
    h = w_hh.shape[1]
    d = w_ih.shape[1]
    wit = w_ih.T
    wht = w_hh.T
    wtop = jnp.concatenate([wit, jnp.zeros((d, h), jnp.float32)], axis=1)
    wtop = jnp.pad(wtop, ((0, dpad - d), (0, 0)))
    wbot = jnp.concatenate(
        [wht[:, :2 * h], jnp.zeros((h, h), jnp.float32), wht[:, 2 * h:]], axis=1)
    bias = jnp.concatenate(
        [b_ih[:2 * h] + b_hh[:2 * h], b_ih[2 * h:], b_hh[2 * h:]]).reshape(1, -1)
    return wtop, wbot, bias


def _run_gru(x_input, w_ih0, w_hh0, b_ih0, b_hh0, w_ih1, w_hh1, b_ih1, b_hh1):
    n, t, d = x_input.shape
    h = w_hh0.shape[1]
    dpad = 8
    xt = jnp.transpose(x_input, (1, 0, 2))
    xt = jnp.pad(xt, ((0, 0), (0, 0), (0, dpad - d)))
    wt0, wb0, b0 = _gate_weights(w_ih0, w_hh0, b_ih0, b_hh0, dpad)
    wt1, wb1, b1 = _gate_weights(w_ih1, w_hh1, b_ih1, b_hh1, h)
    full = lambda shape: pl.BlockSpec(shape, lambda i: (0,) * len(shape))
    return pl.pallas_call(
        _gru_body,
        grid=(t,),
        in_specs=[
            pl.BlockSpec((1, n, dpad), lambda i: (i, 0, 0)),
            full((dpad, 4 * h)), full((h, 4 * h)), full((1, 4 * h)),
            full((h, 4 * h)), full((h, 4 * h)), full((1, 4 * h)),
        ],
        out_specs=pl.BlockSpec((n, h), lambda i: (0, 0)),
        out_shape=jax.ShapeDtypeStruct((n, h), jnp.float32),
        scratch_shapes=[pltpu.VMEM((n, h), jnp.float32),
                        pltpu.VMEM((n, h), jnp.float32)],
    )(xt, wt0, wb0, b0, wt1, wb1, b1)


def _sim_body(ihb_ref, ih_ref, col_ref, diag_ref, pay_ref):
    i = pl.program_id(0)
    ihb = ihb_ref[...]
    ih = ih_ref[...]
    n = ih.shape[0]
    rb = ihb.shape[0]

    s = lax.dot_general(ihb, ih, (((1,), (1,)), ((), ())))
    sq = ih * ih
    ones = jnp.ones((1, ih.shape[1]), jnp.float32)
    cn2 = lax.dot_general(ones, sq, (((1,), (1,)), ((), ())))
    cnorm = jnp.sqrt(cn2)
    rnorm = jnp.sqrt(jnp.sum(ihb * ihb, axis=1, keepdims=True))
    c = s * (1.0 / rnorm) * (1.0 / (cnorm + 1e-6))

    col_ids = lax.broadcasted_iota(jnp.int32, (rb, n), 1)
    row_ids = i * rb + lax.broadcasted_iota(jnp.int32, (rb, 1), 0)
    isdiag = col_ids == row_ids
    diag = jnp.sum(jnp.where(isdiag, c, 0.0), axis=1, keepdims=True)
    cmd = jnp.where(isdiag, 0.0, c)
    value = jnp.max(cmd, axis=1, keepdims=True)
    col = jnp.min(jnp.where(cmd == value, col_ids, n), axis=1, keepdims=True)

    col_ref[0] = col
    diag_ref[0] = diag
    pay_ref[...] = jnp.concatenate(
        [value * ihb, value, jnp.zeros((rb, _PW - ihb.shape[1] - 1), jnp.float32)],
        axis=1)


def _run_sim(ih):
    n, h = ih.shape
    nb = n // _RB
    return pl.pallas_call(
        _sim_body,
        grid=(nb,),
        in_specs=[
            pl.BlockSpec((_RB, h), lambda i: (i, 0)),
            pl.BlockSpec((n, h), lambda i: (0, 0)),
        ],
        out_specs=[
            pl.BlockSpec((1, _RB, 1), lambda i: (i, 0, 0)),
            pl.BlockSpec((1, _RB, 1), lambda i: (i, 0, 0)),
            pl.BlockSpec((_RB, _PW), lambda i: (i, 0)),
        ],
        out_shape=[
            jax.ShapeDtypeStruct((nb, _RB, 1), jnp.int32),
            jax.ShapeDtypeStruct((nb, _RB, 1), jnp.float32),
            jax.ShapeDtypeStruct((n, _PW), jnp.float32),
        ],
    )(ih, ih)


def _run_scatter(col, payload, zeros):
    n = payload.shape[0]
    mesh = plsc.VectorSubcoreMesh(core_axis_name="c", subcore_axis_name="s")
    info = plsc.get_sparse_core_info()
    nc, ns = info.num_cores, info.num_subcores
    rows_per_tile = n // (nc * ns)
    zrows = n // ns

    @functools.partial(
        pl.kernel, mesh=mesh,
        out_type=jax.ShapeDtypeStruct((nc * n, _PW), jnp.float32),
        scratch_types=[
            pltpu.VMEM_SHARED((n, _PW), jnp.float32),
            pltpu.VMEM((rows_per_tile,), jnp.int32),
            pltpu.VMEM((rows_per_tile, _PW), jnp.float32),
        ],
    )
    def k(col_hbm, pay_hbm, z_hbm, out_hbm, acc, idx_v, pay_v):
        c = lax.axis_index("c")
        s = lax.axis_index("s")
        pltpu.sync_copy(z_hbm.at[pl.ds(s * zrows, zrows)],
                        acc.at[pl.ds(s * zrows, zrows)])
        plsc.subcore_barrier()
        base = (c * ns + s) * rows_per_tile
        pltpu.sync_copy(col_hbm.at[pl.ds(base, rows_per_tile)], idx_v)
        pltpu.sync_copy(pay_hbm.at[pl.ds(base, rows_per_tile)], pay_v)
        pltpu.sync_copy(pay_v, acc.at[idx_v], add=True)
        plsc.subcore_barrier()
        pltpu.sync_copy(acc.at[pl.ds(s * zrows, zrows)],
                        out_hbm.at[pl.ds(c * n + s * zrows, zrows)])

    return k(col, payload, zeros)


def _head_body(acc0_ref, acc1_ref, diag_ref, ihb_ref, ih_ref,
               wo_ref, bo_ref, wf_ref, bf_ref, wb_ref, bb_ref,
               wi_ref, bi_ref, wfin_ref, bfin_ref, out_ref):
    ihb = ihb_ref[...]
    ih = ih_ref[...]
    h = ih.shape[1]
    acc = acc0_ref[...] + acc1_ref[...]
    m2 = acc[:, :h]
    colsum = acc[:, h:h + 1]
    diag = diag_ref[0]
    x = m2 + jnp.where(colsum != 0.0, diag, 0.0) * ihb

    s2 = lax.dot_general(x, ih, (((1,), (1,)), ((), ())))
    sq = ih * ih
    ones = jnp.ones((1, h), jnp.float32)
    cn2 = lax.dot_general(ones, sq, (((1,), (1,)), ((), ())))
    cnorm = jnp.sqrt(cn2)
    xnorm = jnp.sqrt(jnp.sum(x * x, axis=1, keepdims=True))
    c2 = s2 * (1.0 / xnorm) * (1.0 / (cnorm + 1e-6))

    m = jnp.max(c2, axis=1, keepdims=True)
    e = jnp.exp(c2 - m)
    p = e * (1.0 / jnp.sum(e, axis=1, keepdims=True))
    agg = lax.dot_general(p, ih, (((1,), (0,)), ((), ())))

    output = jnp.dot(agg, wo_ref[...]) + bo_ref[...]
    fore = jax.nn.leaky_relu(jnp.dot(output, wf_ref[...]) + bf_ref[...], 0.01)
    back = jnp.dot(output, wb_ref[...]) + bb_ref[...]
    ind = jax.nn.leaky_relu(jnp.dot(ihb - back, wi_ref[...]) + bi_ref[...], 0.01)
    out_ref[...] = jnp.dot(fore + ind, wfin_ref[...]) + bfin_ref[...]


def _run_head(acc2, diag, ih, W_out, b_out, W_fore, b_fore, W_back, b_back,
              W_ind, b_ind, W_final, b_final):
    n, h = ih.shape
    nb = n // _RB
    full = lambda shape: pl.BlockSpec(shape, lambda i: (0,) * len(shape))
    return pl.pallas_call(
        _head_body,
        grid=(nb,),
        in_specs=[
            pl.BlockSpec((_RB, _PW), lambda i: (i, 0)),
            pl.BlockSpec((_RB, _PW), lambda i: (i, 0)),
            pl.BlockSpec((1, _RB, 1), lambda i: (i, 0, 0)),
            pl.BlockSpec((_RB, h), lambda i: (i, 0)),
            pl.BlockSpec((n, h), lambda i: (0, 0)),
            full((h, h)), full((1, h)), full((h, h)), full((1, h)),
            full((h, h)), full((1, h)), full((h, h)), full((1, h)),
            full((h, 1)), full((1, 1)),
        ],
        out_specs=pl.BlockSpec((_RB, 1), lambda i: (i, 0)),
        out_shape=jax.ShapeDtypeStruct((n, 1), jnp.float32),
    )(acc2[:n], acc2[n:], diag, ih, ih,
      W_out.T, b_out.reshape(1, -1), W_fore.T, b_fore.reshape(1, -1),
      W_back.T, b_back.reshape(1, -1), W_ind.T, b_ind.reshape(1, -1),
      W_final.T, b_final.reshape(1, -1))


def kernel(x_input, w_ih0, w_hh0, b_ih0, b_hh0, w_ih1, w_hh1, b_ih1, b_hh1,
           W_out, b_out, W_fore, b_fore, W_back, b_back, W_ind, b_ind,
           W_final, b_final):
    n = x_input.shape[0]
    ih = _run_gru(x_input, w_ih0, w_hh0, b_ih0, b_hh0,
                  w_ih1, w_hh1, b_ih1, b_hh1)
    col3, diag3, payload = _run_sim(ih)
    zeros = jnp.zeros((n, _PW), jnp.float32)
    acc2 = _run_scatter(col3.reshape(n), payload, zeros)
    return _run_head(acc2, diag3, ih, W_out, b_out, W_fore, b_fore,
                     W_back, b_back, W_ind, b_ind, W_final, b_final)


# speedup vs baseline: 1.2401x; 1.2401x over previous
import functools

import jax
import jax.numpy as jnp
from jax import lax
from jax.experimental import pallas as pl
from jax.experimental.pallas import tpu as pltpu
from jax.experimental.pallas import tpu_sc as plsc

_RB = 512
_PW = 80


def _gru_body(x_ref, wi0_ref, wh0_ref, bi0_ref, bh0_ref,
              wi1_ref, wh1_ref, bi1_ref, bh1_ref, out_ref, h1_ref, h2_ref):
    t = pl.program_id(0)
    nt = pl.num_programs(0)

    @pl.when(t == 0)
    def _():
        h1_ref[...] = jnp.zeros_like(h1_ref)
        h2_ref[...] = jnp.zeros_like(h2_ref)

    xt = x_ref[0]
    h1 = h1_ref[...]
    h2 = h2_ref[...]

    def cell(xpart, h, wi, wh, bi, bh):
        gi = jnp.dot(xpart, wi) + bi
        gh = jnp.dot(h, wh) + bh
        i_r, i_z, i_n = jnp.split(gi, 3, axis=1)
        h_r, h_z, h_n = jnp.split(gh, 3, axis=1)
        r = jax.nn.sigmoid(i_r + h_r)
        z = jax.nn.sigmoid(i_z + h_z)
        n = jnp.tanh(i_n + r * h_n)
        return (1.0 - z) * n + z * h

    h1n = cell(xt, h1, wi0_ref[...], wh0_ref[...], bi0_ref[...], bh0_ref[...])
    h2n = cell(h1n, h2, wi1_ref[...], wh1_ref[...], bi1_ref[...], bh1_ref[...])
    h1_ref[...] = h1n
    h2_ref[...] = h2n

    @pl.when(t == nt - 1)
    def _():
        out_ref[...] = h2n


def _run_gru(x_input, w_ih0, w_hh0, b_ih0, b_hh0, w_ih1, w_hh1, b_ih1, b_hh1):
    n, t, d = x_input.shape
    h = w_hh0.shape[1]
    dpad = 8
    xt = jnp.transpose(x_input, (1, 0, 2))
    xt = jnp.pad(xt, ((0, 0), (0, 0), (0, dpad - d)))
    wi0 = jnp.pad(w_ih0.T, ((0, dpad - d), (0, 0)))
    full = lambda shape: pl.BlockSpec(shape, lambda i: (0,) * len(shape))
    return pl.pallas_call(
        _gru_body,
        grid=(t,),
        in_specs=[
            pl.BlockSpec((1, n, dpad), lambda i: (i, 0, 0)),
            full((dpad, 3 * h)), full((h, 3 * h)),
            full((1, 3 * h)), full((1, 3 * h)),
            full((h, 3 * h)), full((h, 3 * h)),
            full((1, 3 * h)), full((1, 3 * h)),
        ],
        out_specs=pl.BlockSpec((n, h), lambda i: (0, 0)),
        out_shape=jax.ShapeDtypeStruct((n, h), jnp.float32),
        scratch_shapes=[pltpu.VMEM((n, h), jnp.float32),
                        pltpu.VMEM((n, h), jnp.float32)],
    )(xt, wi0, w_hh0.T, b_ih0.reshape(1, -1), b_hh0.reshape(1, -1),
      w_ih1.T, w_hh1.T, b_ih1.reshape(1, -1), b_hh1.reshape(1, -1))


def _sim_body(ihb_ref, ih_ref, col_ref, diag_ref, pay_ref):
    i = pl.program_id(0)
    ihb = ihb_ref[...]
    ih = ih_ref[...]
    n = ih.shape[0]
    rb = ihb.shape[0]

    s = lax.dot_general(ihb, ih, (((1,), (1,)), ((), ())))
    sq = ih * ih
    ones = jnp.ones((1, ih.shape[1]), jnp.float32)
    cn2 = lax.dot_general(ones, sq, (((1,), (1,)), ((), ())))
    cnorm = jnp.sqrt(cn2)
    rnorm = jnp.sqrt(jnp.sum(ihb * ihb, axis=1, keepdims=True))
    c = s * (1.0 / rnorm) * (1.0 / (cnorm + 1e-6))

    col_ids = lax.broadcasted_iota(jnp.int32, (rb, n), 1)
    row_ids = i * rb + lax.broadcasted_iota(jnp.int32, (rb, 1), 0)
    isdiag = col_ids == row_ids
    diag = jnp.sum(jnp.where(isdiag, c, 0.0), axis=1, keepdims=True)
    cmd = jnp.where(isdiag, 0.0, c)
    value = jnp.max(cmd, axis=1, keepdims=True)
    col = jnp.min(jnp.where(cmd == value, col_ids, n), axis=1, keepdims=True)

    col_ref[0] = col
    diag_ref[0] = diag
    pay_ref[...] = jnp.concatenate(
        [value * ihb, value, jnp.zeros((rb, _PW - ihb.shape[1] - 1), jnp.float32)],
        axis=1)


def _run_sim(ih):
    n, h = ih.shape
    nb = n // _RB
    return pl.pallas_call(
        _sim_body,
        grid=(nb,),
        in_specs=[
            pl.BlockSpec((_RB, h), lambda i: (i, 0)),
            pl.BlockSpec((n, h), lambda i: (0, 0)),
        ],
        out_specs=[
            pl.BlockSpec((1, _RB, 1), lambda i: (i, 0, 0)),
            pl.BlockSpec((1, _RB, 1), lambda i: (i, 0, 0)),
            pl.BlockSpec((_RB, _PW), lambda i: (i, 0)),
        ],
        out_shape=[
            jax.ShapeDtypeStruct((nb, _RB, 1), jnp.int32),
            jax.ShapeDtypeStruct((nb, _RB, 1), jnp.float32),
            jax.ShapeDtypeStruct((n, _PW), jnp.float32),
        ],
    )(ih, ih)


def _run_scatter(col, payload, zeros):
    n = payload.shape[0]
    mesh = plsc.VectorSubcoreMesh(core_axis_name="c", subcore_axis_name="s")
    info = plsc.get_sparse_core_info()
    nc, ns = info.num_cores, info.num_subcores
    rows_per_tile = n // (nc * ns)
    zrows = n // ns

    @functools.partial(
        pl.kernel, mesh=mesh,
        out_type=jax.ShapeDtypeStruct((nc * n, _PW), jnp.float32),
        scratch_types=[
            pltpu.VMEM_SHARED((n, _PW), jnp.float32),
            pltpu.VMEM((rows_per_tile,), jnp.int32),
            pltpu.VMEM((rows_per_tile, _PW), jnp.float32),
        ],
    )
    def k(col_hbm, pay_hbm, z_hbm, out_hbm, acc, idx_v, pay_v):
        c = lax.axis_index("c")
        s = lax.axis_index("s")
        pltpu.sync_copy(z_hbm.at[pl.ds(s * zrows, zrows)],
                        acc.at[pl.ds(s * zrows, zrows)])
        plsc.subcore_barrier()
        base = (c * ns + s) * rows_per_tile
        pltpu.sync_copy(col_hbm.at[pl.ds(base, rows_per_tile)], idx_v)
        pltpu.sync_copy(pay_hbm.at[pl.ds(base, rows_per_tile)], pay_v)
        pltpu.sync_copy(pay_v, acc.at[idx_v], add=True)
        plsc.subcore_barrier()
        pltpu.sync_copy(acc.at[pl.ds(s * zrows, zrows)],
                        out_hbm.at[pl.ds(c * n + s * zrows, zrows)])

    return k(col, payload, zeros)


def _head_body(acc0_ref, acc1_ref, diag_ref, ihb_ref, ih_ref,
               wo_ref, bo_ref, wf_ref, bf_ref, wb_ref, bb_ref,
               wi_ref, bi_ref, wfin_ref, bfin_ref, out_ref):
    ihb = ihb_ref[...]
    ih = ih_ref[...]
    h = ih.shape[1]
    acc = acc0_ref[...] + acc1_ref[...]
    m2 = acc[:, :h]
    colsum = acc[:, h:h + 1]
    diag = diag_ref[0]
    x = m2 + jnp.where(colsum != 0.0, diag, 0.0) * ihb

    s2 = lax.dot_general(x, ih, (((1,), (1,)), ((), ())))
    sq = ih * ih
    ones = jnp.ones((1, h), jnp.float32)
    cn2 = lax.dot_general(ones, sq, (((1,), (1,)), ((), ())))
    cnorm = jnp.sqrt(cn2)
    xnorm = jnp.sqrt(jnp.sum(x * x, axis=1, keepdims=True))
    c2 = s2 * (1.0 / xnorm) * (1.0 / (cnorm + 1e-6))

    m = jnp.max(c2, axis=1, keepdims=True)
    e = jnp.exp(c2 - m)
    p = e * (1.0 / jnp.sum(e, axis=1, keepdims=True))
    agg = lax.dot_general(p, ih, (((1,), (0,)), ((), ())))

    output = jnp.dot(agg, wo_ref[...]) + bo_ref[...]
    fore = jax.nn.leaky_relu(jnp.dot(output, wf_ref[...]) + bf_ref[...], 0.01)
    back = jnp.dot(output, wb_ref[...]) + bb_ref[...]
    ind = jax.nn.leaky_relu(jnp.dot(ihb - back, wi_ref[...]) + bi_ref[...], 0.01)
    out_ref[...] = jnp.dot(fore + ind, wfin_ref[...]) + bfin_ref[...]


def _run_head(acc2, diag, ih, W_out, b_out, W_fore, b_fore, W_back, b_back,
              W_ind, b_ind, W_final, b_final):
    n, h = ih.shape
    nb = n // _RB
    full = lambda shape: pl.BlockSpec(shape, lambda i: (0,) * len(shape))
    return pl.pallas_call(
        _head_body,
        grid=(nb,),
        in_specs=[
            pl.BlockSpec((_RB, _PW), lambda i: (i, 0)),
            pl.BlockSpec((_RB, _PW), lambda i: (i, 0)),
            pl.BlockSpec((1, _RB, 1), lambda i: (i, 0, 0)),
            pl.BlockSpec((_RB, h), lambda i: (i, 0)),
            pl.BlockSpec((n, h), lambda i: (0, 0)),
            full((h, h)), full((1, h)), full((h, h)), full((1, h)),
            full((h, h)), full((1, h)), full((h, h)), full((1, h)),
            full((h, 1)), full((1, 1)),
        ],
        out_specs=pl.BlockSpec((_RB, 1), lambda i: (i, 0)),
        out_shape=jax.ShapeDtypeStruct((n, 1), jnp.float32),
    )(acc2[:n], acc2[n:], diag, ih, ih,
      W_out.T, b_out.reshape(1, -1), W_fore.T, b_fore.reshape(1, -1),
      W_back.T, b_back.reshape(1, -1), W_ind.T, b_ind.reshape(1, -1),
      W_final.T, b_final.reshape(1, -1))


def kernel(x_input, w_ih0, w_hh0, b_ih0, b_hh0, w_ih1, w_hh1, b_ih1, b_hh1,
           W_out, b_out, W_fore, b_fore, W_back, b_back, W_ind, b_ind,
           W_final, b_final):
    n = x_input.shape[0]
    ih = _run_gru(x_input, w_ih0, w_hh0, b_ih0, b_hh0,
                  w_ih1, w_hh1, b_ih1, b_hh1)
    col3, diag3, payload = _run_sim(ih)
    zeros = jnp.zeros((n, _PW), jnp.float32)
    acc2 = _run_scatter(col3.reshape(n), payload, zeros)
    return _run_head(acc2, diag3, ih, W_out, b_out, W_fore, b_fore,
                     W_back, b_back, W_ind, b_ind, W_final, b_final)


# speedup vs baseline: 1.8146x; 1.4633x over previous
import functools

import jax
import jax.numpy as jnp
from jax import lax
from jax.experimental import pallas as pl
from jax.experimental.pallas import tpu as pltpu
from jax.experimental.pallas import tpu_sc as plsc

_RB = 512
_PW = 80


def _gru_body(x_ref, wi0_ref, wh0_ref, bi0_ref, bh0_ref,
              wi1_ref, wh1_ref, bi1_ref, bh1_ref, out_ref, h1_ref, h2_ref):
    t = pl.program_id(0)
    nt = pl.num_programs(0)

    @pl.when(t == 0)
    def _():
        h1_ref[...] = jnp.zeros_like(h1_ref)
        h2_ref[...] = jnp.zeros_like(h2_ref)

    xtt = x_ref[...].reshape(x_ref.shape[1], x_ref.shape[2])
    h1 = h1_ref[...]
    h2 = h2_ref[...]

    def sigmoid(v):
        return 0.5 * jnp.tanh(0.5 * v) + 0.5

    def cell(xpart, h, wi, wh, bi, bh):
        hh = h.shape[0]
        gi = jnp.dot(wi, xpart) + bi
        gh = jnp.dot(wh, h) + bh
        i_r, i_z, i_n = jnp.split(gi, 3, axis=0)
        h_r, h_z, h_n = jnp.split(gh, 3, axis=0)
        r = sigmoid(i_r + h_r)
        z = sigmoid(i_z + h_z)
        n = jnp.tanh(i_n + r * h_n)
        return (1.0 - z) * n + z * h

    h1n = cell(xtt, h1, wi0_ref[...], wh0_ref[...], bi0_ref[...], bh0_ref[...])
    h2n = cell(h1n, h2, wi1_ref[...], wh1_ref[...], bi1_ref[...], bh1_ref[...])
    h1_ref[...] = h1n
    h2_ref[...] = h2n

    @pl.when(t == nt - 1)
    def _():
        out_ref[...] = h2n


def _run_gru(x_input, w_ih0, w_hh0, b_ih0, b_hh0, w_ih1, w_hh1, b_ih1, b_hh1):
    n, t, d = x_input.shape
    h = w_hh0.shape[1]
    xtd = jnp.transpose(x_input.reshape(n, t * d)).reshape(t, d, n)
    full = lambda shape: pl.BlockSpec(shape, lambda i: (0,) * len(shape))
    iht = pl.pallas_call(
        _gru_body,
        grid=(t,),
        in_specs=[
            pl.BlockSpec((1, d, n), lambda i: (i, 0, 0)),
            full((3 * h, d)), full((3 * h, h)),
            full((3 * h, 1)), full((3 * h, 1)),
            full((3 * h, h)), full((3 * h, h)),
            full((3 * h, 1)), full((3 * h, 1)),
        ],
        out_specs=pl.BlockSpec((h, n), lambda i: (0, 0)),
        out_shape=jax.ShapeDtypeStruct((h, n), jnp.float32),
        scratch_shapes=[pltpu.VMEM((h, n), jnp.float32),
                        pltpu.VMEM((h, n), jnp.float32)],
    )(xtd, w_ih0, w_hh0, b_ih0.reshape(-1, 1), b_hh0.reshape(-1, 1),
      w_ih1, w_hh1, b_ih1.reshape(-1, 1), b_hh1.reshape(-1, 1))
    return jnp.transpose(iht)


def _sim_body(ihb_ref, ih_ref, col_ref, diag_ref, pay_ref):
    i = pl.program_id(0)
    ihb = ihb_ref[...]
    ih = ih_ref[...]
    n = ih.shape[0]
    rb = ihb.shape[0]

    cnorm = jnp.sqrt(jnp.sum(ih * ih, axis=1, keepdims=True))
    rnorm = jnp.sqrt(jnp.sum(ihb * ihb, axis=1, keepdims=True))
    ihs = ih * (1.0 / (cnorm + 1e-6))
    ihb_s = ihb * (1.0 / rnorm)
    c = lax.dot_general(ihb_s, ihs, (((1,), (1,)), ((), ())))

    col_ids = lax.broadcasted_iota(jnp.int32, (rb, n), 1)
    row_ids = i * rb + lax.broadcasted_iota(jnp.int32, (rb, 1), 0)
    isdiag = col_ids == row_ids
    diag = jnp.sum(jnp.where(isdiag, c, 0.0), axis=1, keepdims=True)
    cmd = jnp.where(isdiag, 0.0, c)
    value = jnp.max(cmd, axis=1, keepdims=True)
    col = jnp.min(jnp.where(cmd == value, col_ids, n), axis=1, keepdims=True)

    col_ref[0] = col
    diag_ref[0] = diag
    pay_ref[...] = jnp.concatenate(
        [value * ihb, value, jnp.zeros((rb, _PW - ihb.shape[1] - 1), jnp.float32)],
        axis=1)


def _run_sim(ih):
    n, h = ih.shape
    nb = n // _RB
    return pl.pallas_call(
        _sim_body,
        grid=(nb,),
        in_specs=[
            pl.BlockSpec((_RB, h), lambda i: (i, 0)),
            pl.BlockSpec((n, h), lambda i: (0, 0)),
        ],
        out_specs=[
            pl.BlockSpec((1, _RB, 1), lambda i: (i, 0, 0)),
            pl.BlockSpec((1, _RB, 1), lambda i: (i, 0, 0)),
            pl.BlockSpec((_RB, _PW), lambda i: (i, 0)),
        ],
        out_shape=[
            jax.ShapeDtypeStruct((nb, _RB, 1), jnp.int32),
            jax.ShapeDtypeStruct((nb, _RB, 1), jnp.float32),
            jax.ShapeDtypeStruct((n, _PW), jnp.float32),
        ],
    )(ih, ih)


def _run_scatter(col, payload, zeros):
    n = payload.shape[0]
    mesh = plsc.VectorSubcoreMesh(core_axis_name="c", subcore_axis_name="s")
    info = plsc.get_sparse_core_info()
    nc, ns = info.num_cores, info.num_subcores
    rows_per_tile = n // (nc * ns)
    zrows = n // ns

    @functools.partial(
        pl.kernel, mesh=mesh,
        out_type=jax.ShapeDtypeStruct((nc * n, _PW), jnp.float32),
        scratch_types=[
            pltpu.VMEM_SHARED((n, _PW), jnp.float32),
            pltpu.VMEM((rows_per_tile,), jnp.int32),
            pltpu.VMEM((rows_per_tile, _PW), jnp.float32),
        ],
    )
    def k(col_hbm, pay_hbm, z_hbm, out_hbm, acc, idx_v, pay_v):
        c = lax.axis_index("c")
        s = lax.axis_index("s")
        pltpu.sync_copy(z_hbm.at[pl.ds(s * zrows, zrows)],
                        acc.at[pl.ds(s * zrows, zrows)])
        plsc.subcore_barrier()
        base = (c * ns + s) * rows_per_tile
        pltpu.sync_copy(col_hbm.at[pl.ds(base, rows_per_tile)], idx_v)
        pltpu.sync_copy(pay_hbm.at[pl.ds(base, rows_per_tile)], pay_v)
        pltpu.sync_copy(pay_v, acc.at[idx_v], add=True)
        plsc.subcore_barrier()
        pltpu.sync_copy(acc.at[pl.ds(s * zrows, zrows)],
                        out_hbm.at[pl.ds(c * n + s * zrows, zrows)])

    return k(col, payload, zeros)


def _head_body(acc0_ref, acc1_ref, diag_ref, ihb_ref, ih_ref,
               wo_ref, bo_ref, wf_ref, bf_ref, wb_ref, bb_ref,
               wi_ref, bi_ref, wfin_ref, bfin_ref, out_ref):
    ihb = ihb_ref[...]
    ih = ih_ref[...]
    h = ih.shape[1]
    acc = acc0_ref[...] + acc1_ref[...]
    m2 = acc[:, :h]
    colsum = acc[:, h:h + 1]
    diag = diag_ref[0]
    x = m2 + jnp.where(colsum != 0.0, diag, 0.0) * ihb

    cnorm = jnp.sqrt(jnp.sum(ih * ih, axis=1, keepdims=True))
    xnorm = jnp.sqrt(jnp.sum(x * x, axis=1, keepdims=True))
    ihs = ih * (1.0 / (cnorm + 1e-6))
    xs = x * (1.0 / xnorm)
    c2 = lax.dot_general(xs, ihs, (((1,), (1,)), ((), ())))

    m = jnp.max(c2, axis=1, keepdims=True)
    e = jnp.exp(c2 - m)
    p = e * (1.0 / jnp.sum(e, axis=1, keepdims=True))
    agg = lax.dot_general(p, ih, (((1,), (0,)), ((), ())))

    output = jnp.dot(agg, wo_ref[...]) + bo_ref[...]
    fore = jax.nn.leaky_relu(jnp.dot(output, wf_ref[...]) + bf_ref[...], 0.01)
    back = jnp.dot(output, wb_ref[...]) + bb_ref[...]
    ind = jax.nn.leaky_relu(jnp.dot(ihb - back, wi_ref[...]) + bi_ref[...], 0.01)
    out_ref[...] = jnp.dot(fore + ind, wfin_ref[...]) + bfin_ref[...]


def _run_head(acc2, diag, ih, W_out, b_out, W_fore, b_fore, W_back, b_back,
              W_ind, b_ind, W_final, b_final):
    n, h = ih.shape
    nb = n // _RB
    full = lambda shape: pl.BlockSpec(shape, lambda i: (0,) * len(shape))
    return pl.pallas_call(
        _head_body,
        grid=(nb,),
        in_specs=[
            pl.BlockSpec((_RB, _PW), lambda i: (i, 0)),
            pl.BlockSpec((_RB, _PW), lambda i: (i, 0)),
            pl.BlockSpec((1, _RB, 1), lambda i: (i, 0, 0)),
            pl.BlockSpec((_RB, h), lambda i: (i, 0)),
            pl.BlockSpec((n, h), lambda i: (0, 0)),
            full((h, h)), full((1, h)), full((h, h)), full((1, h)),
            full((h, h)), full((1, h)), full((h, h)), full((1, h)),
            full((h, 1)), full((1, 1)),
        ],
        out_specs=pl.BlockSpec((_RB, 1), lambda i: (i, 0)),
        out_shape=jax.ShapeDtypeStruct((n, 1), jnp.float32),
    )(acc2[:n], acc2[n:], diag, ih, ih,
      W_out.T, b_out.reshape(1, -1), W_fore.T, b_fore.reshape(1, -1),
      W_back.T, b_back.reshape(1, -1), W_ind.T, b_ind.reshape(1, -1),
      W_final.T, b_final.reshape(1, -1))


def kernel(x_input, w_ih0, w_hh0, b_ih0, b_hh0, w_ih1, w_hh1, b_ih1, b_hh1,
           W_out, b_out, W_fore, b_fore, W_back, b_back, W_ind, b_ind,
           W_final, b_final):
    n = x_input.shape[0]
    ih = _run_gru(x_input, w_ih0, w_hh0, b_ih0, b_hh0,
                  w_ih1, w_hh1, b_ih1, b_hh1)
    col3, diag3, payload = _run_sim(ih)
    zeros = jnp.zeros((n, _PW), jnp.float32)
    acc2 = _run_scatter(col3.reshape(n), payload, zeros)
    return _run_head(acc2, diag3, ih, W_out, b_out, W_fore, b_fore,
                     W_back, b_back, W_ind, b_ind, W_final, b_final)


# speedup vs baseline: 1.9394x; 1.0688x over previous
import functools

import jax
import jax.numpy as jnp
from jax import lax
from jax.experimental import pallas as pl
from jax.experimental.pallas import tpu as pltpu
from jax.experimental.pallas import tpu_sc as plsc

_RB = 512
_PW = 128


def _gru_body(x_ref, wi0_ref, wh0_ref, b0_ref,
              wi1_ref, wh1_ref, b1_ref, out_ref, h1_ref, h2_ref):
    t = pl.program_id(0)
    nt = pl.num_programs(0)

    @pl.when(t == 0)
    def _():
        h1_ref[...] = jnp.zeros_like(h1_ref)
        h2_ref[...] = jnp.zeros_like(h2_ref)

    xtt = x_ref[...].reshape(x_ref.shape[1], x_ref.shape[2])
    h1 = h1_ref[...]
    h2 = h2_ref[...]

    def cell(xpart, h, wi, wh, b):
        gi = jnp.dot(wi, xpart)
        gh = jnp.dot(wh, h)
        i_r, i_z, i_n = jnp.split(gi, 3, axis=0)
        h_r, h_z, h_n = jnp.split(gh, 3, axis=0)
        hh = h.shape[0]
        b_r = b[0 * hh:1 * hh]
        b_z = b[1 * hh:2 * hh]
        b_in = b[2 * hh:3 * hh]
        b_hn = b[3 * hh:4 * hh]
        r = 0.5 * jnp.tanh(i_r + h_r + b_r) + 0.5
        z = 0.5 * jnp.tanh(i_z + h_z + b_z) + 0.5
        n = jnp.tanh((i_n + b_in) + r * (h_n + b_hn))
        return (1.0 - z) * n + z * h

    h1n = cell(xtt, h1, wi0_ref[...], wh0_ref[...], b0_ref[...])
    h2n = cell(h1n, h2, wi1_ref[...], wh1_ref[...], b1_ref[...])
    h1_ref[...] = h1n
    h2_ref[...] = h2n

    @pl.when(t == nt - 1)
    def _():
        out_ref[...] = h2n


def _gru_params(w_ih, w_hh, b_ih, b_hh):
    h = w_hh.shape[1]
    half = jnp.concatenate([jnp.full((2 * h,), 0.5, jnp.float32),
                            jnp.ones((h,), jnp.float32)])
    wi = w_ih * half[:, None]
    wh = w_hh * half[:, None]
    b = jnp.concatenate([0.5 * (b_ih[:2 * h] + b_hh[:2 * h]),
                         b_ih[2 * h:], b_hh[2 * h:]]).reshape(-1, 1)
    return wi, wh, b


def _run_gru(x_input, w_ih0, w_hh0, b_ih0, b_hh0, w_ih1, w_hh1, b_ih1, b_hh1):
    n, t, d = x_input.shape
    h = w_hh0.shape[1]
    xtd = jnp.transpose(x_input.reshape(n, t * d)).reshape(t, d, n)
    wi0, wh0, b0 = _gru_params(w_ih0, w_hh0, b_ih0, b_hh0)
    wi1, wh1, b1 = _gru_params(w_ih1, w_hh1, b_ih1, b_hh1)
    full = lambda shape: pl.BlockSpec(shape, lambda i: (0,) * len(shape))
    iht = pl.pallas_call(
        _gru_body,
        grid=(t,),
        in_specs=[
            pl.BlockSpec((1, d, n), lambda i: (i, 0, 0)),
            full((3 * h, d)), full((3 * h, h)), full((4 * h, 1)),
            full((3 * h, h)), full((3 * h, h)), full((4 * h, 1)),
        ],
        out_specs=pl.BlockSpec((h, n), lambda i: (0, 0)),
        out_shape=jax.ShapeDtypeStruct((h, n), jnp.float32),
        scratch_shapes=[pltpu.VMEM((h, n), jnp.float32),
                        pltpu.VMEM((h, n), jnp.float32)],
    )(xtd, wi0, wh0, b0, wi1, wh1, b1)
    return jnp.transpose(iht)


def _sim_body(ihb_ref, ih_ref, col_ref, diag_ref, pay_ref):
    i = pl.program_id(0)
    ihb = ihb_ref[...]
    ih = ih_ref[...]
    n = ih.shape[0]
    rb = ihb.shape[0]

    cnorm = jnp.sqrt(jnp.sum(ih * ih, axis=1, keepdims=True))
    rnorm = jnp.sqrt(jnp.sum(ihb * ihb, axis=1, keepdims=True))
    ihs = ih * (1.0 / (cnorm + 1e-6))
    ihb_s = ihb * (1.0 / rnorm)
    c = lax.dot_general(ihb_s, ihs, (((1,), (1,)), ((), ())))

    diag = rnorm * (1.0 / (rnorm + 1e-6))
    col_ids = lax.broadcasted_iota(jnp.int32, (rb, n), 1)
    row_ids = i * rb + lax.broadcasted_iota(jnp.int32, (rb, 1), 0)
    cmd = jnp.where(col_ids == row_ids, 0.0, c)
    value = jnp.max(cmd, axis=1, keepdims=True)
    col = jnp.min(jnp.where(cmd == value, col_ids, n), axis=1, keepdims=True)

    col_ref[0] = col
    diag_ref[0] = diag
    pay_ref[...] = jnp.concatenate(
        [value * ihb, value, jnp.zeros((rb, _PW - ihb.shape[1] - 1), jnp.float32)],
        axis=1)


def _run_sim(ih):
    n, h = ih.shape
    nb = n // _RB
    return pl.pallas_call(
        _sim_body,
        grid=(nb,),
        in_specs=[
            pl.BlockSpec((_RB, h), lambda i: (i, 0)),
            pl.BlockSpec((n, h), lambda i: (0, 0)),
        ],
        out_specs=[
            pl.BlockSpec((1, _RB, 1), lambda i: (i, 0, 0)),
            pl.BlockSpec((1, _RB, 1), lambda i: (i, 0, 0)),
            pl.BlockSpec((_RB, _PW), lambda i: (i, 0)),
        ],
        out_shape=[
            jax.ShapeDtypeStruct((nb, _RB, 1), jnp.int32),
            jax.ShapeDtypeStruct((nb, _RB, 1), jnp.float32),
            jax.ShapeDtypeStruct((n, _PW), jnp.float32),
        ],
    )(ih, ih)


def _run_scatter(col, payload, zeros):
    n = payload.shape[0]
    mesh = plsc.VectorSubcoreMesh(core_axis_name="c", subcore_axis_name="s")
    info = plsc.get_sparse_core_info()
    nc, ns = info.num_cores, info.num_subcores
    rows_per_tile = n // (nc * ns)
    zrows = n // ns

    @functools.partial(
        pl.kernel, mesh=mesh,
        out_type=jax.ShapeDtypeStruct((nc * n, _PW), jnp.float32),
        scratch_types=[
            pltpu.VMEM_SHARED((n, _PW), jnp.float32),
            pltpu.VMEM((rows_per_tile,), jnp.int32),
            pltpu.VMEM((rows_per_tile, _PW), jnp.float32),
        ],
    )
    def k(col_hbm, pay_hbm, z_hbm, out_hbm, acc, idx_v, pay_v):
        c = lax.axis_index("c")
        s = lax.axis_index("s")
        pltpu.sync_copy(z_hbm.at[pl.ds(s * zrows, zrows)],
                        acc.at[pl.ds(s * zrows, zrows)])
        plsc.subcore_barrier()
        base = (c * ns + s) * rows_per_tile
        pltpu.sync_copy(col_hbm.at[pl.ds(base, rows_per_tile)], idx_v)
        pltpu.sync_copy(pay_hbm.at[pl.ds(base, rows_per_tile)], pay_v)
        pltpu.sync_copy(pay_v, acc.at[idx_v], add=True)
        plsc.subcore_barrier()
        pltpu.sync_copy(acc.at[pl.ds(s * zrows, zrows)],
                        out_hbm.at[pl.ds(c * n + s * zrows, zrows)])

    return k(col, payload, zeros)


def _head_body(acc0_ref, acc1_ref, diag_ref, ihb_ref, ih_ref,
               wo_ref, bo_ref, wf_ref, bf_ref, wb_ref, bb_ref,
               wi_ref, bi_ref, wfin_ref, bfin_ref, out_ref):
    ihb = ihb_ref[...]
    ih = ih_ref[...]
    h = ih.shape[1]
    acc = acc0_ref[...] + acc1_ref[...]
    m2 = acc[:, :h]
    colsum = acc[:, h:h + 1]
    diag = diag_ref[0]
    x = m2 + jnp.where(colsum != 0.0, diag, 0.0) * ihb

    cnorm = jnp.sqrt(jnp.sum(ih * ih, axis=1, keepdims=True))
    xnorm = jnp.sqrt(jnp.sum(x * x, axis=1, keepdims=True))
    ihs = ih * (1.0 / (cnorm + 1e-6))
    xs = x * (1.0 / xnorm)
    c2 = lax.dot_general(xs, ihs, (((1,), (1,)), ((), ())))

    m = jnp.max(c2, axis=1, keepdims=True)
    e = jnp.exp(c2 - m)
    agg = lax.dot_general(e, ih, (((1,), (0,)), ((), ())))
    agg = agg * (1.0 / jnp.sum(e, axis=1, keepdims=True))

    output = jnp.dot(agg, wo_ref[...]) + bo_ref[...]
    fore = jax.nn.leaky_relu(jnp.dot(output, wf_ref[...]) + bf_ref[...], 0.01)
    back = jnp.dot(output, wb_ref[...]) + bb_ref[...]
    ind = jax.nn.leaky_relu(jnp.dot(ihb - back, wi_ref[...]) + bi_ref[...], 0.01)
    out_ref[...] = jnp.dot(fore + ind, wfin_ref[...]) + bfin_ref[...]


def _run_head(acc2, diag, ih, W_out, b_out, W_fore, b_fore, W_back, b_back,
              W_ind, b_ind, W_final, b_final):
    n, h = ih.shape
    nb = n // _RB
    full = lambda shape: pl.BlockSpec(shape, lambda i: (0,) * len(shape))
    return pl.pallas_call(
        _head_body,
        grid=(nb,),
        in_specs=[
            pl.BlockSpec((_RB, _PW), lambda i: (i, 0)),
            pl.BlockSpec((_RB, _PW), lambda i: (i, 0)),
            pl.BlockSpec((1, _RB, 1), lambda i: (i, 0, 0)),
            pl.BlockSpec((_RB, h), lambda i: (i, 0)),
            pl.BlockSpec((n, h), lambda i: (0, 0)),
            full((h, h)), full((1, h)), full((h, h)), full((1, h)),
            full((h, h)), full((1, h)), full((h, h)), full((1, h)),
            full((h, 1)), full((1, 1)),
        ],
        out_specs=pl.BlockSpec((_RB, 1), lambda i: (i, 0)),
        out_shape=jax.ShapeDtypeStruct((n, 1), jnp.float32),
    )(acc2[:n], acc2[n:], diag, ih, ih,
      W_out.T, b_out.reshape(1, -1), W_fore.T, b_fore.reshape(1, -1),
      W_back.T, b_back.reshape(1, -1), W_ind.T, b_ind.reshape(1, -1),
      W_final.T, b_final.reshape(1, -1))


def kernel(x_input, w_ih0, w_hh0, b_ih0, b_hh0, w_ih1, w_hh1, b_ih1, b_hh1,
           W_out, b_out, W_fore, b_fore, W_back, b_back, W_ind, b_ind,
           W_final, b_final):
    n = x_input.shape[0]
    ih = _run_gru(x_input, w_ih0, w_hh0, b_ih0, b_hh0,
                  w_ih1, w_hh1, b_ih1, b_hh1)
    col3, diag3, payload = _run_sim(ih)
    zeros = jnp.zeros((n, _PW), jnp.float32)
    acc2 = _run_scatter(col3.reshape(n), payload, zeros)
    return _run_head(acc2, diag3, ih, W_out, b_out, W_fore, b_fore,
                     W_back, b_back, W_ind, b_ind, W_final, b_final)


# speedup vs baseline: 2.2388x; 1.1544x over previous
import functools

import jax
import jax.numpy as jnp
from jax import lax
from jax.experimental import pallas as pl
from jax.experimental.pallas import tpu as pltpu
from jax.experimental.pallas import tpu_sc as plsc

_RB = 512
_PW = 128


def _gru_cell(xpart, h, wi, wh, b):
    gi = jnp.dot(wi, xpart)
    gh = jnp.dot(wh, h)
    i_r, i_z, i_n = jnp.split(gi, 3, axis=0)
    h_r, h_z, h_n = jnp.split(gh, 3, axis=0)
    hh = h.shape[0]
    b_r = b[0 * hh:1 * hh]
    b_z = b[1 * hh:2 * hh]
    b_in = b[2 * hh:3 * hh]
    b_hn = b[3 * hh:4 * hh]
    r = 0.5 * jnp.tanh(i_r + h_r + b_r) + 0.5
    z = 0.5 * jnp.tanh(i_z + h_z + b_z) + 0.5
    n = jnp.tanh((i_n + b_in) + r * (h_n + b_hn))
    return n + z * (h - n)


def _gru_body(xa_ref, xb_ref, wi0_ref, wh0_ref, b0_ref,
              wi1_ref, wh1_ref, b1_ref, out_ref, h1_ref, h2_ref):
    t = pl.program_id(0)
    nt = pl.num_programs(0)

    @pl.when(t == 0)
    def _():
        h1_ref[...] = jnp.zeros_like(h1_ref)
        h2_ref[...] = jnp.zeros_like(h2_ref)

    h1 = h1_ref[...]
    h2 = h2_ref[...]
    wi0, wh0, b0 = wi0_ref[...], wh0_ref[...], b0_ref[...]
    wi1, wh1, b1 = wi1_ref[...], wh1_ref[...], b1_ref[...]
    for x_ref in (xa_ref, xb_ref):
        xtt = x_ref[...].reshape(x_ref.shape[1], x_ref.shape[2])
        h1 = _gru_cell(xtt, h1, wi0, wh0, b0)
        h2 = _gru_cell(h1, h2, wi1, wh1, b1)
    h1_ref[...] = h1
    h2_ref[...] = h2

    @pl.when(t == nt - 1)
    def _():
        out_ref[...] = h2


def _gru_params(w_ih, w_hh, b_ih, b_hh):
    h = w_hh.shape[1]
    half = jnp.concatenate([jnp.full((2 * h,), 0.5, jnp.float32),
                            jnp.ones((h,), jnp.float32)])
    wi = w_ih * half[:, None]
    wh = w_hh * half[:, None]
    b = jnp.concatenate([0.5 * (b_ih[:2 * h] + b_hh[:2 * h]),
                         b_ih[2 * h:], b_hh[2 * h:]]).reshape(-1, 1)
    return wi, wh, b


def _run_gru(x_input, w_ih0, w_hh0, b_ih0, b_hh0, w_ih1, w_hh1, b_ih1, b_hh1):
    n, t, d = x_input.shape
    h = w_hh0.shape[1]
    xtd = jnp.transpose(x_input.reshape(n, t * d)).reshape(t, d, n)
    wi0, wh0, b0 = _gru_params(w_ih0, w_hh0, b_ih0, b_hh0)
    wi1, wh1, b1 = _gru_params(w_ih1, w_hh1, b_ih1, b_hh1)
    full = lambda shape: pl.BlockSpec(shape, lambda i: (0,) * len(shape))
    iht = pl.pallas_call(
        _gru_body,
        grid=(t // 2,),
        in_specs=[
            pl.BlockSpec((1, d, n), lambda i: (2 * i, 0, 0)),
            pl.BlockSpec((1, d, n), lambda i: (2 * i + 1, 0, 0)),
            full((3 * h, d)), full((3 * h, h)), full((4 * h, 1)),
            full((3 * h, h)), full((3 * h, h)), full((4 * h, 1)),
        ],
        out_specs=pl.BlockSpec((h, n), lambda i: (0, 0)),
        out_shape=jax.ShapeDtypeStruct((h, n), jnp.float32),
        scratch_shapes=[pltpu.VMEM((h, n), jnp.float32),
                        pltpu.VMEM((h, n), jnp.float32)],
    )(xtd, xtd, wi0, wh0, b0, wi1, wh1, b1)
    return jnp.transpose(iht)


def _sim_body(ihb_ref, ih_ref, col_ref, diag_ref, pay_ref):
    i = pl.program_id(0)
    ihb = ihb_ref[...]
    ih = ih_ref[...]
    n = ih.shape[0]
    rb = ihb.shape[0]

    cnorm = jnp.sqrt(jnp.sum(ih * ih, axis=1, keepdims=True))
    rnorm = jnp.sqrt(jnp.sum(ihb * ihb, axis=1, keepdims=True))
    ihs = ih * (1.0 / (cnorm + 1e-6))
    ihb_s = ihb * (1.0 / rnorm)
    c = lax.dot_general(ihb_s, ihs, (((1,), (1,)), ((), ())))

    diag = rnorm * (1.0 / (rnorm + 1e-6))
    col_ids = lax.broadcasted_iota(jnp.int32, (rb, n), 1)
    row_ids = i * rb + lax.broadcasted_iota(jnp.int32, (rb, 1), 0)
    cmd = jnp.where(col_ids == row_ids, 0.0, c)
    value = jnp.max(cmd, axis=1, keepdims=True)
    col = jnp.min(jnp.where(cmd == value, col_ids, n), axis=1, keepdims=True)

    col_ref[0] = col
    diag_ref[0] = diag
    pay_ref[...] = jnp.concatenate(
        [value * ihb, value, jnp.zeros((rb, _PW - ihb.shape[1] - 1), jnp.float32)],
        axis=1)


def _run_sim(ih):
    n, h = ih.shape
    nb = n // _RB
    return pl.pallas_call(
        _sim_body,
        grid=(nb,),
        in_specs=[
            pl.BlockSpec((_RB, h), lambda i: (i, 0)),
            pl.BlockSpec((n, h), lambda i: (0, 0)),
        ],
        out_specs=[
            pl.BlockSpec((1, _RB, 1), lambda i: (i, 0, 0)),
            pl.BlockSpec((1, _RB, 1), lambda i: (i, 0, 0)),
            pl.BlockSpec((_RB, _PW), lambda i: (i, 0)),
        ],
        out_shape=[
            jax.ShapeDtypeStruct((nb, _RB, 1), jnp.int32),
            jax.ShapeDtypeStruct((nb, _RB, 1), jnp.float32),
            jax.ShapeDtypeStruct((n, _PW), jnp.float32),
        ],
    )(ih, ih)


def _run_scatter(col, payload, zeros):
    n = payload.shape[0]
    mesh = plsc.VectorSubcoreMesh(core_axis_name="c", subcore_axis_name="s")
    info = plsc.get_sparse_core_info()
    nc, ns = info.num_cores, info.num_subcores
    rows_per_tile = n // (nc * ns)
    zrows = n // ns

    @functools.partial(
        pl.kernel, mesh=mesh,
        out_type=jax.ShapeDtypeStruct((nc * n, _PW), jnp.float32),
        scratch_types=[
            pltpu.VMEM_SHARED((n, _PW), jnp.float32),
            pltpu.VMEM((rows_per_tile,), jnp.int32),
            pltpu.VMEM((rows_per_tile, _PW), jnp.float32),
        ],
    )
    def k(col_hbm, pay_hbm, z_hbm, out_hbm, acc, idx_v, pay_v):
        c = lax.axis_index("c")
        s = lax.axis_index("s")
        pltpu.sync_copy(z_hbm.at[pl.ds(s * zrows, zrows)],
                        acc.at[pl.ds(s * zrows, zrows)])
        plsc.subcore_barrier()
        base = (c * ns + s) * rows_per_tile
        pltpu.sync_copy(col_hbm.at[pl.ds(base, rows_per_tile)], idx_v)
        pltpu.sync_copy(pay_hbm.at[pl.ds(base, rows_per_tile)], pay_v)
        pltpu.sync_copy(pay_v, acc.at[idx_v], add=True)
        plsc.subcore_barrier()
        pltpu.sync_copy(acc.at[pl.ds(s * zrows, zrows)],
                        out_hbm.at[pl.ds(c * n + s * zrows, zrows)])

    return k(col, payload, zeros)


def _head_body(acc0_ref, acc1_ref, diag_ref, ihb_ref, ih_ref,
               wo_ref, bo_ref, wf_ref, bf_ref, wb_ref, bb_ref,
               wi_ref, bi_ref, wfin_ref, bfin_ref, out_ref):
    ihb = ihb_ref[...]
    ih = ih_ref[...]
    h = ih.shape[1]
    acc = acc0_ref[...] + acc1_ref[...]
    m2 = acc[:, :h]
    colsum = acc[:, h:h + 1]
    diag = diag_ref[0]
    x = m2 + jnp.where(colsum != 0.0, diag, 0.0) * ihb

    cnorm = jnp.sqrt(jnp.sum(ih * ih, axis=1, keepdims=True))
    xnorm = jnp.sqrt(jnp.sum(x * x, axis=1, keepdims=True))
    ihs = ih * (1.0 / (cnorm + 1e-6))
    xs = x * (1.0 / xnorm)
    c2 = lax.dot_general(xs, ihs, (((1,), (1,)), ((), ())))

    e = jnp.exp(c2)
    agg = lax.dot_general(e, ih, (((1,), (0,)), ((), ())))
    agg = agg * (1.0 / jnp.sum(e, axis=1, keepdims=True))

    output = jnp.dot(agg, wo_ref[...]) + bo_ref[...]
    fore = jax.nn.leaky_relu(jnp.dot(output, wf_ref[...]) + bf_ref[...], 0.01)
    back = jnp.dot(output, wb_ref[...]) + bb_ref[...]
    ind = jax.nn.leaky_relu(jnp.dot(ihb - back, wi_ref[...]) + bi_ref[...], 0.01)
    out_ref[...] = jnp.dot(fore + ind, wfin_ref[...]) + bfin_ref[...]


def _run_head(acc2, diag, ih, W_out, b_out, W_fore, b_fore, W_back, b_back,
              W_ind, b_ind, W_final, b_final):
    n, h = ih.shape
    nb = n // _RB
    full = lambda shape: pl.BlockSpec(shape, lambda i: (0,) * len(shape))
    return pl.pallas_call(
        _head_body,
        grid=(nb,),
        in_specs=[
            pl.BlockSpec((_RB, _PW), lambda i: (i, 0)),
            pl.BlockSpec((_RB, _PW), lambda i: (i, 0)),
            pl.BlockSpec((1, _RB, 1), lambda i: (i, 0, 0)),
            pl.BlockSpec((_RB, h), lambda i: (i, 0)),
            pl.BlockSpec((n, h), lambda i: (0, 0)),
            full((h, h)), full((1, h)), full((h, h)), full((1, h)),
            full((h, h)), full((1, h)), full((h, h)), full((1, h)),
            full((h, 1)), full((1, 1)),
        ],
        out_specs=pl.BlockSpec((_RB, 1), lambda i: (i, 0)),
        out_shape=jax.ShapeDtypeStruct((n, 1), jnp.float32),
    )(acc2[:n], acc2[n:], diag, ih, ih,
      W_out.T, b_out.reshape(1, -1), W_fore.T, b_fore.reshape(1, -1),
      W_back.T, b_back.reshape(1, -1), W_ind.T, b_ind.reshape(1, -1),
      W_final.T, b_final.reshape(1, -1))


def kernel(x_input, w_ih0, w_hh0, b_ih0, b_hh0, w_ih1, w_hh1, b_ih1, b_hh1,
           W_out, b_out, W_fore, b_fore, W_back, b_back, W_ind, b_ind,
           W_final, b_final):
    n = x_input.shape[0]
    ih = _run_gru(x_input, w_ih0, w_hh0, b_ih0, b_hh0,
                  w_ih1, w_hh1, b_ih1, b_hh1)
    col3, diag3, payload = _run_sim(ih)
    zeros = jnp.zeros((n, _PW), jnp.float32)
    acc2 = _run_scatter(col3.reshape(n), payload, zeros)
    return _run_head(acc2, diag3, ih, W_out, b_out, W_fore, b_fore,
                     W_back, b_back, W_ind, b_ind, W_final, b_final)


# speedup vs baseline: 2.3113x; 1.0324x over previous
import functools

import jax
import jax.numpy as jnp
from jax import lax
from jax.experimental import pallas as pl
from jax.experimental.pallas import tpu as pltpu
from jax.experimental.pallas import tpu_sc as plsc

_RB = 512
_PW = 128


def _gru_cell(xpart, h, wi, wh, b):
    gi = jnp.dot(wi, xpart)
    gh = jnp.dot(wh, h)
    i_r, i_z, i_n = jnp.split(gi, 3, axis=0)
    h_r, h_z, h_n = jnp.split(gh, 3, axis=0)
    hh = h.shape[0]
    b_r = b[0 * hh:1 * hh]
    b_z = b[1 * hh:2 * hh]
    b_in = b[2 * hh:3 * hh]
    b_hn = b[3 * hh:4 * hh]
    r = 0.5 * jnp.tanh(i_r + h_r + b_r) + 0.5
    z = 0.5 * jnp.tanh(i_z + h_z + b_z) + 0.5
    n = jnp.tanh((i_n + b_in) + r * (h_n + b_hn))
    return n + z * (h - n)


def _gru_body(xa_ref, xb_ref, xc_ref, xd_ref, wi0_ref, wh0_ref, b0_ref,
              wi1_ref, wh1_ref, b1_ref, out_ref, h1_ref, h2_ref):
    t = pl.program_id(0)
    nt = pl.num_programs(0)

    @pl.when(t == 0)
    def _():
        h1_ref[...] = jnp.zeros_like(h1_ref)
        h2_ref[...] = jnp.zeros_like(h2_ref)

    h1 = h1_ref[...]
    h2 = h2_ref[...]
    wi0, wh0, b0 = wi0_ref[...], wh0_ref[...], b0_ref[...]
    wi1, wh1, b1 = wi1_ref[...], wh1_ref[...], b1_ref[...]
    for x_ref in (xa_ref, xb_ref, xc_ref, xd_ref):
        xtt = x_ref[...].reshape(x_ref.shape[1], x_ref.shape[2])
        h1 = _gru_cell(xtt, h1, wi0, wh0, b0)
        h2 = _gru_cell(h1, h2, wi1, wh1, b1)
    h1_ref[...] = h1
    h2_ref[...] = h2

    @pl.when(t == nt - 1)
    def _():
        out_ref[...] = h2


def _gru_params(w_ih, w_hh, b_ih, b_hh):
    h = w_hh.shape[1]
    half = jnp.concatenate([jnp.full((2 * h,), 0.5, jnp.float32),
                            jnp.ones((h,), jnp.float32)])
    wi = w_ih * half[:, None]
    wh = w_hh * half[:, None]
    b = jnp.concatenate([0.5 * (b_ih[:2 * h] + b_hh[:2 * h]),
                         b_ih[2 * h:], b_hh[2 * h:]]).reshape(-1, 1)
    return wi, wh, b


def _run_gru(x_input, w_ih0, w_hh0, b_ih0, b_hh0, w_ih1, w_hh1, b_ih1, b_hh1):
    n, t, d = x_input.shape
    h = w_hh0.shape[1]
    xtd = jnp.transpose(x_input.reshape(n, t * d)).reshape(t, d, n)
    wi0, wh0, b0 = _gru_params(w_ih0, w_hh0, b_ih0, b_hh0)
    wi1, wh1, b1 = _gru_params(w_ih1, w_hh1, b_ih1, b_hh1)
    full = lambda shape: pl.BlockSpec(shape, lambda i: (0,) * len(shape))
    iht = pl.pallas_call(
        _gru_body,
        grid=(t // 4,),
        in_specs=[
            pl.BlockSpec((1, d, n), lambda i: (4 * i, 0, 0)),
            pl.BlockSpec((1, d, n), lambda i: (4 * i + 1, 0, 0)),
            pl.BlockSpec((1, d, n), lambda i: (4 * i + 2, 0, 0)),
            pl.BlockSpec((1, d, n), lambda i: (4 * i + 3, 0, 0)),
            full((3 * h, d)), full((3 * h, h)), full((4 * h, 1)),
            full((3 * h, h)), full((3 * h, h)), full((4 * h, 1)),
        ],
        out_specs=pl.BlockSpec((h, n), lambda i: (0, 0)),
        out_shape=jax.ShapeDtypeStruct((h, n), jnp.float32),
        scratch_shapes=[pltpu.VMEM((h, n), jnp.float32),
                        pltpu.VMEM((h, n), jnp.float32)],
    )(xtd, xtd, xtd, xtd, wi0, wh0, b0, wi1, wh1, b1)
    return jnp.transpose(iht)


def _sim_body(ihb_ref, ih_ref, col_ref, diag_ref, pay_ref):
    i = pl.program_id(0)
    ihb = ihb_ref[...]
    ih = ih_ref[...]
    n = ih.shape[0]
    rb = ihb.shape[0]

    cnorm = jnp.sqrt(jnp.sum(ih * ih, axis=1, keepdims=True))
    rnorm = jnp.sqrt(jnp.sum(ihb * ihb, axis=1, keepdims=True))
    ihs = ih * (1.0 / (cnorm + 1e-6))
    ihb_s = ihb * (1.0 / rnorm)
    c = lax.dot_general(ihb_s, ihs, (((1,), (1,)), ((), ())))

    diag = rnorm * (1.0 / (rnorm + 1e-6))
    col_ids = lax.broadcasted_iota(jnp.int32, (rb, n), 1)
    row_ids = i * rb + lax.broadcasted_iota(jnp.int32, (rb, 1), 0)
    cmd = jnp.where(col_ids == row_ids, 0.0, c)
    value = jnp.max(cmd, axis=1, keepdims=True)
    col = jnp.min(jnp.where(cmd == value, col_ids, n), axis=1, keepdims=True)

    col_ref[0] = col
    diag_ref[0] = diag
    pay_ref[...] = jnp.concatenate(
        [value * ihb, value, jnp.zeros((rb, _PW - ihb.shape[1] - 1), jnp.float32)],
        axis=1)


def _run_sim(ih):
    n, h = ih.shape
    nb = n // _RB
    return pl.pallas_call(
        _sim_body,
        grid=(nb,),
        in_specs=[
            pl.BlockSpec((_RB, h), lambda i: (i, 0)),
            pl.BlockSpec((n, h), lambda i: (0, 0)),
        ],
        out_specs=[
            pl.BlockSpec((1, _RB, 1), lambda i: (i, 0, 0)),
            pl.BlockSpec((1, _RB, 1), lambda i: (i, 0, 0)),
            pl.BlockSpec((_RB, _PW), lambda i: (i, 0)),
        ],
        out_shape=[
            jax.ShapeDtypeStruct((nb, _RB, 1), jnp.int32),
            jax.ShapeDtypeStruct((nb, _RB, 1), jnp.float32),
            jax.ShapeDtypeStruct((n, _PW), jnp.float32),
        ],
    )(ih, ih)


def _run_scatter(col, payload, zeros):
    n = payload.shape[0]
    mesh = plsc.VectorSubcoreMesh(core_axis_name="c", subcore_axis_name="s")
    info = plsc.get_sparse_core_info()
    nc, ns = info.num_cores, info.num_subcores
    rows_per_tile = n // (nc * ns)
    zrows = n // ns

    @functools.partial(
        pl.kernel, mesh=mesh,
        out_type=jax.ShapeDtypeStruct((nc * n, _PW), jnp.float32),
        scratch_types=[
            pltpu.VMEM_SHARED((n, _PW), jnp.float32),
            pltpu.VMEM((rows_per_tile,), jnp.int32),
            pltpu.VMEM((rows_per_tile, _PW), jnp.float32),
        ],
    )
    def k(col_hbm, pay_hbm, z_hbm, out_hbm, acc, idx_v, pay_v):
        c = lax.axis_index("c")
        s = lax.axis_index("s")
        pltpu.sync_copy(z_hbm.at[pl.ds(s * zrows, zrows)],
                        acc.at[pl.ds(s * zrows, zrows)])
        plsc.subcore_barrier()
        base = (c * ns + s) * rows_per_tile
        pltpu.sync_copy(col_hbm.at[pl.ds(base, rows_per_tile)], idx_v)
        pltpu.sync_copy(pay_hbm.at[pl.ds(base, rows_per_tile)], pay_v)
        pltpu.sync_copy(pay_v, acc.at[idx_v], add=True)
        plsc.subcore_barrier()
        pltpu.sync_copy(acc.at[pl.ds(s * zrows, zrows)],
                        out_hbm.at[pl.ds(c * n + s * zrows, zrows)])

    return k(col, payload, zeros)


def _head_body(acc0_ref, acc1_ref, diag_ref, ihb_ref, ih_ref,
               wo_ref, bo_ref, wf_ref, bf_ref, wb_ref, bb_ref,
               wi_ref, bi_ref, wfin_ref, bfin_ref, out_ref):
    ihb = ihb_ref[...]
    ih = ih_ref[...]
    h = ih.shape[1]
    acc = acc0_ref[...] + acc1_ref[...]
    m2 = acc[:, :h]
    colsum = acc[:, h:h + 1]
    diag = diag_ref[0]
    x = m2 + jnp.where(colsum != 0.0, diag, 0.0) * ihb

    cnorm = jnp.sqrt(jnp.sum(ih * ih, axis=1, keepdims=True))
    xnorm = jnp.sqrt(jnp.sum(x * x, axis=1, keepdims=True))
    ihs = ih * (1.0 / (cnorm + 1e-6))
    xs = x * (1.0 / xnorm)
    c2 = lax.dot_general(xs, ihs, (((1,), (1,)), ((), ())))

    e = jnp.exp(c2)
    agg = lax.dot_general(e, ih, (((1,), (0,)), ((), ())))
    agg = agg * (1.0 / jnp.sum(e, axis=1, keepdims=True))

    output = jnp.dot(agg, wo_ref[...]) + bo_ref[...]
    fore = jax.nn.leaky_relu(jnp.dot(output, wf_ref[...]) + bf_ref[...], 0.01)
    back = jnp.dot(output, wb_ref[...]) + bb_ref[...]
    ind = jax.nn.leaky_relu(jnp.dot(ihb - back, wi_ref[...]) + bi_ref[...], 0.01)
    out_ref[...] = jnp.dot(fore + ind, wfin_ref[...]) + bfin_ref[...]


def _run_head(acc2, diag, ih, W_out, b_out, W_fore, b_fore, W_back, b_back,
              W_ind, b_ind, W_final, b_final):
    n, h = ih.shape
    nb = n // _RB
    full = lambda shape: pl.BlockSpec(shape, lambda i: (0,) * len(shape))
    return pl.pallas_call(
        _head_body,
        grid=(nb,),
        in_specs=[
            pl.BlockSpec((_RB, _PW), lambda i: (i, 0)),
            pl.BlockSpec((_RB, _PW), lambda i: (i + n // _RB, 0)),
            pl.BlockSpec((1, _RB, 1), lambda i: (i, 0, 0)),
            pl.BlockSpec((_RB, h), lambda i: (i, 0)),
            pl.BlockSpec((n, h), lambda i: (0, 0)),
            full((h, h)), full((1, h)), full((h, h)), full((1, h)),
            full((h, h)), full((1, h)), full((h, h)), full((1, h)),
            full((h, 1)), full((1, 1)),
        ],
        out_specs=pl.BlockSpec((_RB, 1), lambda i: (i, 0)),
        out_shape=jax.ShapeDtypeStruct((n, 1), jnp.float32),
    )(acc2, acc2, diag, ih, ih,
      W_out.T, b_out.reshape(1, -1), W_fore.T, b_fore.reshape(1, -1),
      W_back.T, b_back.reshape(1, -1), W_ind.T, b_ind.reshape(1, -1),
      W_final.T, b_final.reshape(1, -1))


def kernel(x_input, w_ih0, w_hh0, b_ih0, b_hh0, w_ih1, w_hh1, b_ih1, b_hh1,
           W_out, b_out, W_fore, b_fore, W_back, b_back, W_ind, b_ind,
           W_final, b_final):
    n = x_input.shape[0]
    ih = _run_gru(x_input, w_ih0, w_hh0, b_ih0, b_hh0,
                  w_ih1, w_hh1, b_ih1, b_hh1)
    col3, diag3, payload = _run_sim(ih)
    zeros = jnp.zeros((n, _PW), jnp.float32)
    acc2 = _run_scatter(col3.reshape(n), payload, zeros)
    return _run_head(acc2, diag3, ih, W_out, b_out, W_fore, b_fore,
                     W_back, b_back, W_ind, b_ind, W_final, b_final)


# speedup vs baseline: 2.3426x; 1.0136x over previous
import functools

import jax
import jax.numpy as jnp
from jax import lax
from jax.experimental import pallas as pl
from jax.experimental.pallas import tpu as pltpu
from jax.experimental.pallas import tpu_sc as plsc

_RB = 512
_PW = 128


def _gru_cell(xpart, h, wi, wh, b):
    gi = jnp.dot(wi, xpart)
    gh = jnp.dot(wh, h)
    i_r, i_z, i_n = jnp.split(gi, 3, axis=0)
    h_r, h_z, h_n = jnp.split(gh, 3, axis=0)
    hh = h.shape[0]
    b_r = b[0 * hh:1 * hh]
    b_z = b[1 * hh:2 * hh]
    b_in = b[2 * hh:3 * hh]
    b_hn = b[3 * hh:4 * hh]
    r = 0.5 * jnp.tanh(i_r + h_r + b_r) + 0.5
    z = 0.5 * jnp.tanh(i_z + h_z + b_z) + 0.5
    n = jnp.tanh((i_n + b_in) + r * (h_n + b_hn))
    return n + z * (h - n)


def _gru_body(xa_ref, xb_ref, xc_ref, xd_ref, wi0_ref, wh0_ref, b0_ref,
              wi1_ref, wh1_ref, b1_ref, out_ref, h1_ref, h2_ref):
    t = pl.program_id(0)
    nt = pl.num_programs(0)

    @pl.when(t == 0)
    def _():
        h1_ref[...] = jnp.zeros_like(h1_ref)
        h2_ref[...] = jnp.zeros_like(h2_ref)

    h1 = h1_ref[...]
    h2 = h2_ref[...]
    wi0, wh0, b0 = wi0_ref[...], wh0_ref[...], b0_ref[...]
    wi1, wh1, b1 = wi1_ref[...], wh1_ref[...], b1_ref[...]
    for x_ref in (xa_ref, xb_ref, xc_ref, xd_ref):
        xtt = x_ref[...].reshape(x_ref.shape[1], x_ref.shape[2])
        h1 = _gru_cell(xtt, h1, wi0, wh0, b0)
        h2 = _gru_cell(h1, h2, wi1, wh1, b1)
    h1_ref[...] = h1
    h2_ref[...] = h2

    @pl.when(t == nt - 1)
    def _():
        out_ref[...] = h2.T


def _gru_params(w_ih, w_hh, b_ih, b_hh):
    h = w_hh.shape[1]
    half = jnp.concatenate([jnp.full((2 * h,), 0.5, jnp.float32),
                            jnp.ones((h,), jnp.float32)])
    wi = w_ih * half[:, None]
    wh = w_hh * half[:, None]
    b = jnp.concatenate([0.5 * (b_ih[:2 * h] + b_hh[:2 * h]),
                         b_ih[2 * h:], b_hh[2 * h:]]).reshape(-1, 1)
    return wi, wh, b


def _run_gru(x_input, w_ih0, w_hh0, b_ih0, b_hh0, w_ih1, w_hh1, b_ih1, b_hh1):
    n, t, d = x_input.shape
    h = w_hh0.shape[1]
    xtd = jnp.transpose(x_input.reshape(n, t * d)).reshape(t, d, n)
    wi0, wh0, b0 = _gru_params(w_ih0, w_hh0, b_ih0, b_hh0)
    wi1, wh1, b1 = _gru_params(w_ih1, w_hh1, b_ih1, b_hh1)
    full = lambda shape: pl.BlockSpec(shape, lambda i: (0,) * len(shape))
    iht = pl.pallas_call(
        _gru_body,
        grid=(t // 4,),
        in_specs=[
            pl.BlockSpec((1, d, n), lambda i: (4 * i, 0, 0)),
            pl.BlockSpec((1, d, n), lambda i: (4 * i + 1, 0, 0)),
            pl.BlockSpec((1, d, n), lambda i: (4 * i + 2, 0, 0)),
            pl.BlockSpec((1, d, n), lambda i: (4 * i + 3, 0, 0)),
            full((3 * h, d)), full((3 * h, h)), full((4 * h, 1)),
            full((3 * h, h)), full((3 * h, h)), full((4 * h, 1)),
        ],
        out_specs=pl.BlockSpec((n, h), lambda i: (0, 0)),
        out_shape=jax.ShapeDtypeStruct((n, h), jnp.float32),
        scratch_shapes=[pltpu.VMEM((h, n), jnp.float32),
                        pltpu.VMEM((h, n), jnp.float32)],
    )(xtd, xtd, xtd, xtd, wi0, wh0, b0, wi1, wh1, b1)
    return iht


def _sim_body(ihb_ref, ih_ref, col_ref, diag_ref, pay_ref):
    i = pl.program_id(0)
    ihb = ihb_ref[...]
    ih = ih_ref[...]
    n = ih.shape[0]
    rb = ihb.shape[0]

    cnorm = jnp.sqrt(jnp.sum(ih * ih, axis=1, keepdims=True))
    rnorm = jnp.sqrt(jnp.sum(ihb * ihb, axis=1, keepdims=True))
    ihs = ih * (1.0 / (cnorm + 1e-6))
    ihb_s = ihb * (1.0 / rnorm)
    c = lax.dot_general(ihb_s, ihs, (((1,), (1,)), ((), ())))

    diag = rnorm * (1.0 / (rnorm + 1e-6))
    col_ids = lax.broadcasted_iota(jnp.int32, (rb, n), 1)
    row_ids = i * rb + lax.broadcasted_iota(jnp.int32, (rb, 1), 0)
    cmd = jnp.where(col_ids == row_ids, 0.0, c)
    value = jnp.max(cmd, axis=1, keepdims=True)
    col = jnp.min(jnp.where(cmd == value, col_ids, n), axis=1, keepdims=True)

    col_ref[0] = col
    diag_ref[0] = diag
    pay_ref[...] = jnp.concatenate(
        [value * ihb, value, jnp.zeros((rb, _PW - ihb.shape[1] - 1), jnp.float32)],
        axis=1)


def _run_sim(ih):
    n, h = ih.shape
    nb = n // _RB
    return pl.pallas_call(
        _sim_body,
        grid=(nb,),
        in_specs=[
            pl.BlockSpec((_RB, h), lambda i: (i, 0)),
            pl.BlockSpec((n, h), lambda i: (0, 0)),
        ],
        out_specs=[
            pl.BlockSpec((1, _RB, 1), lambda i: (i, 0, 0)),
            pl.BlockSpec((1, _RB, 1), lambda i: (i, 0, 0)),
            pl.BlockSpec((_RB, _PW), lambda i: (i, 0)),
        ],
        out_shape=[
            jax.ShapeDtypeStruct((nb, _RB, 1), jnp.int32),
            jax.ShapeDtypeStruct((nb, _RB, 1), jnp.float32),
            jax.ShapeDtypeStruct((n, _PW), jnp.float32),
        ],
    )(ih, ih)


def _run_scatter(col, payload, zeros):
    n = payload.shape[0]
    mesh = plsc.VectorSubcoreMesh(core_axis_name="c", subcore_axis_name="s")
    info = plsc.get_sparse_core_info()
    nc, ns = info.num_cores, info.num_subcores
    rows_per_tile = n // (nc * ns)
    zrows = n // ns

    @functools.partial(
        pl.kernel, mesh=mesh,
        out_type=jax.ShapeDtypeStruct((nc * n, _PW), jnp.float32),
        scratch_types=[
            pltpu.VMEM_SHARED((n, _PW), jnp.float32),
            pltpu.VMEM((rows_per_tile,), jnp.int32),
            pltpu.VMEM((rows_per_tile, _PW), jnp.float32),
        ],
    )
    def k(col_hbm, pay_hbm, z_hbm, out_hbm, acc, idx_v, pay_v):
        c = lax.axis_index("c")
        s = lax.axis_index("s")
        pltpu.sync_copy(z_hbm.at[pl.ds(s * zrows, zrows)],
                        acc.at[pl.ds(s * zrows, zrows)])
        plsc.subcore_barrier()
        base = (c * ns + s) * rows_per_tile
        pltpu.sync_copy(col_hbm.at[pl.ds(base, rows_per_tile)], idx_v)
        pltpu.sync_copy(pay_hbm.at[pl.ds(base, rows_per_tile)], pay_v)
        pltpu.sync_copy(pay_v, acc.at[idx_v], add=True)
        plsc.subcore_barrier()
        pltpu.sync_copy(acc.at[pl.ds(s * zrows, zrows)],
                        out_hbm.at[pl.ds(c * n + s * zrows, zrows)])

    return k(col, payload, zeros)


def _head_body(acc0_ref, acc1_ref, diag_ref, ihb_ref, ih_ref,
               wo_ref, bo_ref, wf_ref, bf_ref, wb_ref, bb_ref,
               wi_ref, bi_ref, wfin_ref, bfin_ref, out_ref):
    ihb = ihb_ref[...]
    ih = ih_ref[...]
    h = ih.shape[1]
    acc = acc0_ref[...] + acc1_ref[...]
    m2 = acc[:, :h]
    colsum = acc[:, h:h + 1]
    diag = diag_ref[0]
    x = m2 + jnp.where(colsum != 0.0, diag, 0.0) * ihb

    cnorm = jnp.sqrt(jnp.sum(ih * ih, axis=1, keepdims=True))
    xnorm = jnp.sqrt(jnp.sum(x * x, axis=1, keepdims=True))
    ihs = ih * (1.0 / (cnorm + 1e-6))
    xs = x * (1.0 / xnorm)
    c2 = lax.dot_general(xs, ihs, (((1,), (1,)), ((), ())))

    e = jnp.exp(c2)
    agg = lax.dot_general(e, ih, (((1,), (0,)), ((), ())))
    agg = agg * (1.0 / jnp.sum(e, axis=1, keepdims=True))

    output = jnp.dot(agg, wo_ref[...]) + bo_ref[...]
    fore = jax.nn.leaky_relu(jnp.dot(output, wf_ref[...]) + bf_ref[...], 0.01)
    back = jnp.dot(output, wb_ref[...]) + bb_ref[...]
    ind = jax.nn.leaky_relu(jnp.dot(ihb - back, wi_ref[...]) + bi_ref[...], 0.01)
    out_ref[...] = jnp.dot(fore + ind, wfin_ref[...]) + bfin_ref[...]


def _run_head(acc2, diag, ih, W_out, b_out, W_fore, b_fore, W_back, b_back,
              W_ind, b_ind, W_final, b_final):
    n, h = ih.shape
    nb = n // _RB
    full = lambda shape: pl.BlockSpec(shape, lambda i: (0,) * len(shape))
    return pl.pallas_call(
        _head_body,
        grid=(nb,),
        in_specs=[
            pl.BlockSpec((_RB, _PW), lambda i: (i, 0)),
            pl.BlockSpec((_RB, _PW), lambda i: (i + n // _RB, 0)),
            pl.BlockSpec((1, _RB, 1), lambda i: (i, 0, 0)),
            pl.BlockSpec((_RB, h), lambda i: (i, 0)),
            pl.BlockSpec((n, h), lambda i: (0, 0)),
            full((h, h)), full((1, h)), full((h, h)), full((1, h)),
            full((h, h)), full((1, h)), full((h, h)), full((1, h)),
            full((h, 1)), full((1, 1)),
        ],
        out_specs=pl.BlockSpec((_RB, 1), lambda i: (i, 0)),
        out_shape=jax.ShapeDtypeStruct((n, 1), jnp.float32),
    )(acc2, acc2, diag, ih, ih,
      W_out.T, b_out.reshape(1, -1), W_fore.T, b_fore.reshape(1, -1),
      W_back.T, b_back.reshape(1, -1), W_ind.T, b_ind.reshape(1, -1),
      W_final.T, b_final.reshape(1, -1))


def kernel(x_input, w_ih0, w_hh0, b_ih0, b_hh0, w_ih1, w_hh1, b_ih1, b_hh1,
           W_out, b_out, W_fore, b_fore, W_back, b_back, W_ind, b_ind,
           W_final, b_final):
    n = x_input.shape[0]
    ih = _run_gru(x_input, w_ih0, w_hh0, b_ih0, b_hh0,
                  w_ih1, w_hh1, b_ih1, b_hh1)
    col3, diag3, payload = _run_sim(ih)
    zeros = jnp.zeros((n, _PW), jnp.float32)
    acc2 = _run_scatter(col3.reshape(n), payload, zeros)
    return _run_head(acc2, diag3, ih, W_out, b_out, W_fore, b_fore,
                     W_back, b_back, W_ind, b_ind, W_final, b_final)


# speedup vs baseline: 2.3946x; 1.0222x over previous
import functools

import jax
import jax.numpy as jnp
from jax import lax
from jax.experimental import pallas as pl
from jax.experimental.pallas import tpu as pltpu
from jax.experimental.pallas import tpu_sc as plsc

_RB = 1024
_PW = 72


def _gru_cell(xpart, h, wi, wh, b):
    gi = jnp.dot(wi, xpart)
    gh = jnp.dot(wh, h)
    i_r, i_z, i_n = jnp.split(gi, 3, axis=0)
    h_r, h_z, h_n = jnp.split(gh, 3, axis=0)
    hh = h.shape[0]
    b_r = b[0 * hh:1 * hh]
    b_z = b[1 * hh:2 * hh]
    b_in = b[2 * hh:3 * hh]
    b_hn = b[3 * hh:4 * hh]
    r = 0.5 * jnp.tanh(i_r + h_r + b_r) + 0.5
    z = 0.5 * jnp.tanh(i_z + h_z + b_z) + 0.5
    n = jnp.tanh((i_n + b_in) + r * (h_n + b_hn))
    return n + z * (h - n)


def _gru_body(xa_ref, xb_ref, xc_ref, xd_ref, wi0_ref, wh0_ref, b0_ref,
              wi1_ref, wh1_ref, b1_ref, out_ref, h1_ref, h2_ref):
    t = pl.program_id(0)
    nt = pl.num_programs(0)

    @pl.when(t == 0)
    def _():
        h1_ref[...] = jnp.zeros_like(h1_ref)
        h2_ref[...] = jnp.zeros_like(h2_ref)

    h1 = h1_ref[...]
    h2 = h2_ref[...]
    wi0, wh0, b0 = wi0_ref[...], wh0_ref[...], b0_ref[...]
    wi1, wh1, b1 = wi1_ref[...], wh1_ref[...], b1_ref[...]
    for x_ref in (xa_ref, xb_ref, xc_ref, xd_ref):
        xtt = x_ref[...].reshape(x_ref.shape[1], x_ref.shape[2])
        h1 = _gru_cell(xtt, h1, wi0, wh0, b0)
        h2 = _gru_cell(h1, h2, wi1, wh1, b1)
    h1_ref[...] = h1
    h2_ref[...] = h2

    @pl.when(t == nt - 1)
    def _():
        out_ref[...] = h2.T


def _gru_params(w_ih, w_hh, b_ih, b_hh):
    h = w_hh.shape[1]
    half = jnp.concatenate([jnp.full((2 * h,), 0.5, jnp.float32),
                            jnp.ones((h,), jnp.float32)])
    wi = w_ih * half[:, None]
    wh = w_hh * half[:, None]
    b = jnp.concatenate([0.5 * (b_ih[:2 * h] + b_hh[:2 * h]),
                         b_ih[2 * h:], b_hh[2 * h:]]).reshape(-1, 1)
    return wi, wh, b


def _run_gru(x_input, w_ih0, w_hh0, b_ih0, b_hh0, w_ih1, w_hh1, b_ih1, b_hh1):
    n, t, d = x_input.shape
    h = w_hh0.shape[1]
    xtd = jnp.transpose(x_input.reshape(n, t * d)).reshape(t, d, n)
    wi0, wh0, b0 = _gru_params(w_ih0, w_hh0, b_ih0, b_hh0)
    wi1, wh1, b1 = _gru_params(w_ih1, w_hh1, b_ih1, b_hh1)
    full = lambda shape: pl.BlockSpec(shape, lambda i: (0,) * len(shape))
    iht = pl.pallas_call(
        _gru_body,
        grid=(t // 4,),
        in_specs=[
            pl.BlockSpec((1, d, n), lambda i: (4 * i, 0, 0)),
            pl.BlockSpec((1, d, n), lambda i: (4 * i + 1, 0, 0)),
            pl.BlockSpec((1, d, n), lambda i: (4 * i + 2, 0, 0)),
            pl.BlockSpec((1, d, n), lambda i: (4 * i + 3, 0, 0)),
            full((3 * h, d)), full((3 * h, h)), full((4 * h, 1)),
            full((3 * h, h)), full((3 * h, h)), full((4 * h, 1)),
        ],
        out_specs=pl.BlockSpec((n, h), lambda i: (0, 0)),
        out_shape=jax.ShapeDtypeStruct((n, h), jnp.float32),
        scratch_shapes=[pltpu.VMEM((h, n), jnp.float32),
                        pltpu.VMEM((h, n), jnp.float32)],
    )(xtd, xtd, xtd, xtd, wi0, wh0, b0, wi1, wh1, b1)
    return iht


def _sim_body(ihb_ref, ih_ref, col_ref, diag_ref, pay_ref):
    i = pl.program_id(0)
    ihb = ihb_ref[...]
    ih = ih_ref[...]
    n = ih.shape[0]
    rb = ihb.shape[0]

    cnorm = jnp.sqrt(jnp.sum(ih * ih, axis=1, keepdims=True))
    rnorm = jnp.sqrt(jnp.sum(ihb * ihb, axis=1, keepdims=True))
    ihs = ih * (1.0 / (cnorm + 1e-6))
    ihb_s = ihb * (1.0 / rnorm)
    c = lax.dot_general(ihb_s, ihs, (((1,), (1,)), ((), ())))

    diag = rnorm * (1.0 / (rnorm + 1e-6))
    col_ids = lax.broadcasted_iota(jnp.int32, (rb, n), 1)
    row_ids = i * rb + lax.broadcasted_iota(jnp.int32, (rb, 1), 0)
    cmd = jnp.where(col_ids == row_ids, 0.0, c)
    value = jnp.max(cmd, axis=1, keepdims=True)
    col = jnp.min(jnp.where(cmd == value, col_ids, n), axis=1, keepdims=True)

    col_ref[0] = col
    diag_ref[0] = diag
    pay_ref[...] = jnp.concatenate(
        [value * ihb, value, jnp.zeros((rb, _PW - ihb.shape[1] - 1), jnp.float32)],
        axis=1)


def _run_sim(ih):
    n, h = ih.shape
    nb = n // _RB
    return pl.pallas_call(
        _sim_body,
        grid=(nb,),
        in_specs=[
            pl.BlockSpec((_RB, h), lambda i: (i, 0)),
            pl.BlockSpec((n, h), lambda i: (0, 0)),
        ],
        out_specs=[
            pl.BlockSpec((1, _RB, 1), lambda i: (i, 0, 0)),
            pl.BlockSpec((1, _RB, 1), lambda i: (i, 0, 0)),
            pl.BlockSpec((_RB, _PW), lambda i: (i, 0)),
        ],
        out_shape=[
            jax.ShapeDtypeStruct((nb, _RB, 1), jnp.int32),
            jax.ShapeDtypeStruct((nb, _RB, 1), jnp.float32),
            jax.ShapeDtypeStruct((n, _PW), jnp.float32),
        ],
    )(ih, ih)


def _run_scatter(col, payload, zeros):
    n = payload.shape[0]
    mesh = plsc.VectorSubcoreMesh(core_axis_name="c", subcore_axis_name="s")
    info = plsc.get_sparse_core_info()
    nc, ns = info.num_cores, info.num_subcores
    rows_per_tile = n // (nc * ns)
    zrows = n // ns

    @functools.partial(
        pl.kernel, mesh=mesh,
        out_type=jax.ShapeDtypeStruct((nc * n, _PW), jnp.float32),
        scratch_types=[
            pltpu.VMEM_SHARED((n, _PW), jnp.float32),
            pltpu.VMEM((rows_per_tile,), jnp.int32),
            pltpu.VMEM((rows_per_tile, _PW), jnp.float32),
        ],
    )
    def k(col_hbm, pay_hbm, z_hbm, out_hbm, acc, idx_v, pay_v):
        c = lax.axis_index("c")
        s = lax.axis_index("s")
        pltpu.sync_copy(z_hbm.at[pl.ds(s * zrows, zrows)],
                        acc.at[pl.ds(s * zrows, zrows)])
        plsc.subcore_barrier()
        base = (c * ns + s) * rows_per_tile
        pltpu.sync_copy(col_hbm.at[pl.ds(base, rows_per_tile)], idx_v)
        pltpu.sync_copy(pay_hbm.at[pl.ds(base, rows_per_tile)], pay_v)
        pltpu.sync_copy(pay_v, acc.at[idx_v], add=True)
        plsc.subcore_barrier()
        pltpu.sync_copy(acc.at[pl.ds(s * zrows, zrows)],
                        out_hbm.at[pl.ds(c * n + s * zrows, zrows)])

    return k(col, payload, zeros)


def _head_body(acc0_ref, acc1_ref, diag_ref, ihb_ref, ih_ref,
               wo_ref, bo_ref, wf_ref, bf_ref, wb_ref, bb_ref,
               wi_ref, bi_ref, wfin_ref, bfin_ref, out_ref):
    ihb = ihb_ref[...]
    ih = ih_ref[...]
    h = ih.shape[1]
    acc = acc0_ref[...] + acc1_ref[...]
    m2 = acc[:, :h]
    colsum = acc[:, h:h + 1]
    diag = diag_ref[0]
    x = m2 + jnp.where(colsum != 0.0, diag, 0.0) * ihb

    cnorm = jnp.sqrt(jnp.sum(ih * ih, axis=1, keepdims=True))
    xnorm = jnp.sqrt(jnp.sum(x * x, axis=1, keepdims=True))
    ihs = ih * (1.0 / (cnorm + 1e-6))
    xs = x * (1.0 / xnorm)
    c2 = lax.dot_general(xs, ihs, (((1,), (1,)), ((), ())))

    e = jnp.exp(c2)
    agg = lax.dot_general(e, ih, (((1,), (0,)), ((), ())))
    agg = agg * (1.0 / jnp.sum(e, axis=1, keepdims=True))

    output = jnp.dot(agg, wo_ref[...]) + bo_ref[...]
    fore = jax.nn.leaky_relu(jnp.dot(output, wf_ref[...]) + bf_ref[...], 0.01)
    back = jnp.dot(output, wb_ref[...]) + bb_ref[...]
    ind = jax.nn.leaky_relu(jnp.dot(ihb - back, wi_ref[...]) + bi_ref[...], 0.01)
    out_ref[...] = jnp.dot(fore + ind, wfin_ref[...]) + bfin_ref[...]


def _run_head(acc2, diag, ih, W_out, b_out, W_fore, b_fore, W_back, b_back,
              W_ind, b_ind, W_final, b_final):
    n, h = ih.shape
    nb = n // _RB
    full = lambda shape: pl.BlockSpec(shape, lambda i: (0,) * len(shape))
    return pl.pallas_call(
        _head_body,
        grid=(nb,),
        in_specs=[
            pl.BlockSpec((_RB, _PW), lambda i: (i, 0)),
            pl.BlockSpec((_RB, _PW), lambda i: (i + n // _RB, 0)),
            pl.BlockSpec((1, _RB, 1), lambda i: (i, 0, 0)),
            pl.BlockSpec((_RB, h), lambda i: (i, 0)),
            pl.BlockSpec((n, h), lambda i: (0, 0)),
            full((h, h)), full((1, h)), full((h, h)), full((1, h)),
            full((h, h)), full((1, h)), full((h, h)), full((1, h)),
            full((h, 1)), full((1, 1)),
        ],
        out_specs=pl.BlockSpec((_RB, 1), lambda i: (i, 0)),
        out_shape=jax.ShapeDtypeStruct((n, 1), jnp.float32),
    )(acc2, acc2, diag, ih, ih,
      W_out.T, b_out.reshape(1, -1), W_fore.T, b_fore.reshape(1, -1),
      W_back.T, b_back.reshape(1, -1), W_ind.T, b_ind.reshape(1, -1),
      W_final.T, b_final.reshape(1, -1))


def kernel(x_input, w_ih0, w_hh0, b_ih0, b_hh0, w_ih1, w_hh1, b_ih1, b_hh1,
           W_out, b_out, W_fore, b_fore, W_back, b_back, W_ind, b_ind,
           W_final, b_final):
    n = x_input.shape[0]
    ih = _run_gru(x_input, w_ih0, w_hh0, b_ih0, b_hh0,
                  w_ih1, w_hh1, b_ih1, b_hh1)
    col3, diag3, payload = _run_sim(ih)
    zeros = jnp.zeros((n, _PW), jnp.float32)
    acc2 = _run_scatter(col3.reshape(n), payload, zeros)
    return _run_head(acc2, diag3, ih, W_out, b_out, W_fore, b_fore,
                     W_back, b_back, W_ind, b_ind, W_final, b_final)


# speedup vs baseline: 2.4126x; 1.0075x over previous
import functools

import jax
import jax.numpy as jnp
from jax import lax
from jax.experimental import pallas as pl
from jax.experimental.pallas import tpu as pltpu
from jax.experimental.pallas import tpu_sc as plsc

_RB = 1024
_PW = 72


def _gru_cell(xpart, h, wi, wh, b):
    gi = jnp.dot(wi, xpart)
    gh = jnp.dot(wh, h)
    i_r, i_z, i_n = jnp.split(gi, 3, axis=0)
    h_r, h_z, h_n = jnp.split(gh, 3, axis=0)
    hh = h.shape[0]
    b_r = b[0 * hh:1 * hh]
    b_z = b[1 * hh:2 * hh]
    b_in = b[2 * hh:3 * hh]
    b_hn = b[3 * hh:4 * hh]
    r = 0.5 * jnp.tanh(i_r + h_r + b_r) + 0.5
    z = 0.5 * jnp.tanh(i_z + h_z + b_z) + 0.5
    n = jnp.tanh((i_n + b_in) + r * (h_n + b_hn))
    return n + z * (h - n)


def _gru_body(xa_ref, xb_ref, xc_ref, xd_ref, xe_ref, xf_ref,
              wi0_ref, wh0_ref, b0_ref,
              wi1_ref, wh1_ref, b1_ref, out_ref, h1_ref, h2_ref):
    t = pl.program_id(0)
    nt = pl.num_programs(0)

    @pl.when(t == 0)
    def _():
        h1_ref[...] = jnp.zeros_like(h1_ref)
        h2_ref[...] = jnp.zeros_like(h2_ref)

    h1 = h1_ref[...]
    h2 = h2_ref[...]
    wi0, wh0, b0 = wi0_ref[...], wh0_ref[...], b0_ref[...]
    wi1, wh1, b1 = wi1_ref[...], wh1_ref[...], b1_ref[...]
    for x_ref in (xa_ref, xb_ref, xc_ref, xd_ref, xe_ref, xf_ref):
        xtt = x_ref[...].reshape(x_ref.shape[1], x_ref.shape[2])
        h1 = _gru_cell(xtt, h1, wi0, wh0, b0)
        h2 = _gru_cell(h1, h2, wi1, wh1, b1)
    h1_ref[...] = h1
    h2_ref[...] = h2

    @pl.when(t == nt - 1)
    def _():
        out_ref[...] = h2.T


def _gru_params(w_ih, w_hh, b_ih, b_hh):
    h = w_hh.shape[1]
    half = jnp.concatenate([jnp.full((2 * h,), 0.5, jnp.float32),
                            jnp.ones((h,), jnp.float32)])
    wi = w_ih * half[:, None]
    wh = w_hh * half[:, None]
    b = jnp.concatenate([0.5 * (b_ih[:2 * h] + b_hh[:2 * h]),
                         b_ih[2 * h:], b_hh[2 * h:]]).reshape(-1, 1)
    return wi, wh, b


def _run_gru(x_input, w_ih0, w_hh0, b_ih0, b_hh0, w_ih1, w_hh1, b_ih1, b_hh1):
    n, t, d = x_input.shape
    h = w_hh0.shape[1]
    xtd = jnp.transpose(x_input.reshape(n, t * d)).reshape(t, d, n)
    wi0, wh0, b0 = _gru_params(w_ih0, w_hh0, b_ih0, b_hh0)
    wi1, wh1, b1 = _gru_params(w_ih1, w_hh1, b_ih1, b_hh1)
    full = lambda shape: pl.BlockSpec(shape, lambda i: (0,) * len(shape))
    iht = pl.pallas_call(
        _gru_body,
        grid=(t // 6,),
        in_specs=[
            pl.BlockSpec((1, d, n), lambda i: (6 * i, 0, 0)),
            pl.BlockSpec((1, d, n), lambda i: (6 * i + 1, 0, 0)),
            pl.BlockSpec((1, d, n), lambda i: (6 * i + 2, 0, 0)),
            pl.BlockSpec((1, d, n), lambda i: (6 * i + 3, 0, 0)),
            pl.BlockSpec((1, d, n), lambda i: (6 * i + 4, 0, 0)),
            pl.BlockSpec((1, d, n), lambda i: (6 * i + 5, 0, 0)),
            full((3 * h, d)), full((3 * h, h)), full((4 * h, 1)),
            full((3 * h, h)), full((3 * h, h)), full((4 * h, 1)),
        ],
        out_specs=pl.BlockSpec((n, h), lambda i: (0, 0)),
        out_shape=jax.ShapeDtypeStruct((n, h), jnp.float32),
        scratch_shapes=[pltpu.VMEM((h, n), jnp.float32),
                        pltpu.VMEM((h, n), jnp.float32)],
    )(xtd, xtd, xtd, xtd, xtd, xtd, wi0, wh0, b0, wi1, wh1, b1)
    return iht


def _sim_body(ihb_ref, ih_ref, col_ref, diag_ref, pay_ref):
    i = pl.program_id(0)
    ihb = ihb_ref[...]
    ih = ih_ref[...]
    n = ih.shape[0]
    rb = ihb.shape[0]

    cnorm = jnp.sqrt(jnp.sum(ih * ih, axis=1, keepdims=True))
    rnorm = jnp.sqrt(jnp.sum(ihb * ihb, axis=1, keepdims=True))
    ihs = ih * (1.0 / (cnorm + 1e-6))
    ihb_s = ihb * (1.0 / rnorm)
    c = lax.dot_general(ihb_s, ihs, (((1,), (1,)), ((), ())))

    diag = rnorm * (1.0 / (rnorm + 1e-6))
    col_ids = lax.broadcasted_iota(jnp.int32, (rb, n), 1)
    row_ids = i * rb + lax.broadcasted_iota(jnp.int32, (rb, 1), 0)
    cmd = jnp.where(col_ids == row_ids, 0.0, c)
    value = jnp.max(cmd, axis=1, keepdims=True)
    col = jnp.min(jnp.where(cmd == value, col_ids, n), axis=1, keepdims=True)

    col_ref[0] = col
    diag_ref[0] = diag
    pay_ref[...] = jnp.concatenate(
        [value * ihb, value, jnp.zeros((rb, _PW - ihb.shape[1] - 1), jnp.float32)],
        axis=1)


def _run_sim(ih):
    n, h = ih.shape
    nb = n // _RB
    return pl.pallas_call(
        _sim_body,
        grid=(nb,),
        in_specs=[
            pl.BlockSpec((_RB, h), lambda i: (i, 0)),
            pl.BlockSpec((n, h), lambda i: (0, 0)),
        ],
        out_specs=[
            pl.BlockSpec((1, _RB, 1), lambda i: (i, 0, 0)),
            pl.BlockSpec((1, _RB, 1), lambda i: (i, 0, 0)),
            pl.BlockSpec((_RB, _PW), lambda i: (i, 0)),
        ],
        out_shape=[
            jax.ShapeDtypeStruct((nb, _RB, 1), jnp.int32),
            jax.ShapeDtypeStruct((nb, _RB, 1), jnp.float32),
            jax.ShapeDtypeStruct((n, _PW), jnp.float32),
        ],
    )(ih, ih)


def _run_scatter(col, payload, zeros):
    n = payload.shape[0]
    mesh = plsc.VectorSubcoreMesh(core_axis_name="c", subcore_axis_name="s")
    info = plsc.get_sparse_core_info()
    nc, ns = info.num_cores, info.num_subcores
    rows_per_tile = n // (nc * ns)
    zrows = n // ns

    @functools.partial(
        pl.kernel, mesh=mesh,
        out_type=jax.ShapeDtypeStruct((nc * n, _PW), jnp.float32),
        scratch_types=[
            pltpu.VMEM_SHARED((n, _PW), jnp.float32),
            pltpu.VMEM((rows_per_tile,), jnp.int32),
            pltpu.VMEM((rows_per_tile, _PW), jnp.float32),
        ],
    )
    def k(col_hbm, pay_hbm, z_hbm, out_hbm, acc, idx_v, pay_v):
        c = lax.axis_index("c")
        s = lax.axis_index("s")
        pltpu.sync_copy(z_hbm.at[pl.ds(s * zrows, zrows)],
                        acc.at[pl.ds(s * zrows, zrows)])
        plsc.subcore_barrier()
        base = (c * ns + s) * rows_per_tile
        pltpu.sync_copy(col_hbm.at[pl.ds(base, rows_per_tile)], idx_v)
        pltpu.sync_copy(pay_hbm.at[pl.ds(base, rows_per_tile)], pay_v)
        pltpu.sync_copy(pay_v, acc.at[idx_v], add=True)
        plsc.subcore_barrier()
        pltpu.sync_copy(acc.at[pl.ds(s * zrows, zrows)],
                        out_hbm.at[pl.ds(c * n + s * zrows, zrows)])

    return k(col, payload, zeros)


def _head_body(acc0_ref, acc1_ref, diag_ref, ihb_ref, ih_ref,
               wo_ref, bo_ref, wf_ref, bf_ref, wb_ref, bb_ref,
               wi_ref, bi_ref, wfin_ref, bfin_ref, out_ref):
    ihb = ihb_ref[...]
    ih = ih_ref[...]
    h = ih.shape[1]
    acc = acc0_ref[...] + acc1_ref[...]
    m2 = acc[:, :h]
    colsum = acc[:, h:h + 1]
    diag = diag_ref[0]
    x = m2 + jnp.where(colsum != 0.0, diag, 0.0) * ihb

    cnorm = jnp.sqrt(jnp.sum(ih * ih, axis=1, keepdims=True))
    xnorm = jnp.sqrt(jnp.sum(x * x, axis=1, keepdims=True))
    ihs = ih * (1.0 / (cnorm + 1e-6))
    xs = x * (1.0 / xnorm)
    c2 = lax.dot_general(xs, ihs, (((1,), (1,)), ((), ())))

    e = jnp.exp(c2)
    agg = lax.dot_general(e, ih, (((1,), (0,)), ((), ())))
    agg = agg * (1.0 / jnp.sum(e, axis=1, keepdims=True))

    output = jnp.dot(agg, wo_ref[...]) + bo_ref[...]
    fore = jax.nn.leaky_relu(jnp.dot(output, wf_ref[...]) + bf_ref[...], 0.01)
    back = jnp.dot(output, wb_ref[...]) + bb_ref[...]
    ind = jax.nn.leaky_relu(jnp.dot(ihb - back, wi_ref[...]) + bi_ref[...], 0.01)
    out_ref[...] = jnp.dot(fore + ind, wfin_ref[...]) + bfin_ref[...]


def _run_head(acc2, diag, ih, W_out, b_out, W_fore, b_fore, W_back, b_back,
              W_ind, b_ind, W_final, b_final):
    n, h = ih.shape
    nb = n // _RB
    full = lambda shape: pl.BlockSpec(shape, lambda i: (0,) * len(shape))
    return pl.pallas_call(
        _head_body,
        grid=(nb,),
        in_specs=[
            pl.BlockSpec((_RB, _PW), lambda i: (i, 0)),
            pl.BlockSpec((_RB, _PW), lambda i: (i + n // _RB, 0)),
            pl.BlockSpec((1, _RB, 1), lambda i: (i, 0, 0)),
            pl.BlockSpec((_RB, h), lambda i: (i, 0)),
            pl.BlockSpec((n, h), lambda i: (0, 0)),
            full((h, h)), full((1, h)), full((h, h)), full((1, h)),
            full((h, h)), full((1, h)), full((h, h)), full((1, h)),
            full((h, 1)), full((1, 1)),
        ],
        out_specs=pl.BlockSpec((_RB, 1), lambda i: (i, 0)),
        out_shape=jax.ShapeDtypeStruct((n, 1), jnp.float32),
    )(acc2, acc2, diag, ih, ih,
      W_out.T, b_out.reshape(1, -1), W_fore.T, b_fore.reshape(1, -1),
      W_back.T, b_back.reshape(1, -1), W_ind.T, b_ind.reshape(1, -1),
      W_final.T, b_final.reshape(1, -1))


def kernel(x_input, w_ih0, w_hh0, b_ih0, b_hh0, w_ih1, w_hh1, b_ih1, b_hh1,
           W_out, b_out, W_fore, b_fore, W_back, b_back, W_ind, b_ind,
           W_final, b_final):
    n = x_input.shape[0]
    ih = _run_gru(x_input, w_ih0, w_hh0, b_ih0, b_hh0,
                  w_ih1, w_hh1, b_ih1, b_hh1)
    col3, diag3, payload = _run_sim(ih)
    zeros = jnp.zeros((n, _PW), jnp.float32)
    acc2 = _run_scatter(col3.reshape(n), payload, zeros)
    return _run_head(acc2, diag3, ih, W_out, b_out, W_fore, b_fore,
                     W_back, b_back, W_ind, b_ind, W_final, b_final)
